# Initial kernel scaffold; baseline (speedup 1.0000x reference)
import functools

import jax
import jax.numpy as jnp
from jax import lax
from jax.experimental import pallas as pl
from jax.experimental.pallas import tpu as pltpu
from jax.experimental.pallas import tpu_sc as plsc

N = 10000
E = 320000
NC = 2
NS = 16
LANES = 16
EPS = E // NS
CHUNK = 80
NCHUNKS = EPS // CHUNK
RPS = N // NS
DEG_W = 16

H1, H2, H3, H_FC = 64, 128, 256, 256


def _pq_call(x, w, b2d, deg, h_in, h_out):
    hh = h_out // 2
    bn = 1000
    use_deg = deg is not None

    def body(*refs):
        if use_deg:
            x_ref, w_ref, b_ref, deg_ref, p_ref, q_ref = refs
        else:
            x_ref, w_ref, b_ref, p_ref, q_ref = refs
        xb = x_ref[...]
        if use_deg:
            xb = xb * (1.0 / jnp.maximum(deg_ref[...][:, :1], 1.0))
        wa = w_ref[:h_in, :]
        wb = w_ref[h_in:, :]
        p_ref[0] = (
            jnp.dot(xb, wa - wb, preferred_element_type=jnp.float32)
            + b_ref[...]
        )
        q_ref[0] = jnp.dot(xb, wb, preferred_element_type=jnp.float32)

    in_specs = [
        pl.BlockSpec((bn, h_in), lambda c, i: (i, 0)),
        pl.BlockSpec((2 * h_in, hh), lambda c, i: (0, c)),
        pl.BlockSpec((1, hh), lambda c, i: (0, c)),
    ]
    args = [x, w, b2d]
    if use_deg:
        in_specs.append(pl.BlockSpec((bn, DEG_W), lambda c, i: (i, 0)))
        args.append(deg)
    out_specs = [
        pl.BlockSpec((1, bn, hh), lambda c, i: (c, i, 0)),
        pl.BlockSpec((1, bn, hh), lambda c, i: (c, i, 0)),
    ]
    out_shape = [
        jax.ShapeDtypeStruct((NC, N, hh), jnp.float32),
        jax.ShapeDtypeStruct((NC, N, hh), jnp.float32),
    ]
    return pl.pallas_call(
        body,
        grid=(NC, N // bn),
        in_specs=in_specs,
        out_specs=out_specs,
        out_shape=out_shape,
    )(*args)


def _edge_call(p, q, dst3, src3, zeros, with_deg):
    hh = p.shape[2]
    mesh = plsc.VectorSubcoreMesh(core_axis_name="c", subcore_axis_name="s")

    out_type = [jax.ShapeDtypeStruct((NC, N, hh), jnp.float32)]
    scratch = [
        pltpu.VMEM((NCHUNKS, CHUNK), jnp.int32),
        pltpu.VMEM((NCHUNKS, CHUNK), jnp.int32),
        pltpu.VMEM((CHUNK, hh), jnp.float32),
        pltpu.VMEM((CHUNK, hh), jnp.float32),
        pltpu.VMEM_SHARED((N, hh), jnp.float32),
        pltpu.SemaphoreType.DMA,
        pltpu.SemaphoreType.DMA,
    ]
    if with_deg:
        out_type.append(jax.ShapeDtypeStruct((N, DEG_W), jnp.float32))
        scratch += [
            pltpu.VMEM((CHUNK, DEG_W), jnp.float32),
            pltpu.VMEM_SHARED((N, DEG_W), jnp.float32),
        ]

    @functools.partial(
        pl.kernel,
        out_type=tuple(out_type) if with_deg else out_type[0],
        mesh=mesh,
        scratch_types=scratch,
    )
    def k(*refs):
        if with_deg:
            (p_hbm, q_hbm, dst_hbm, src_hbm, z_hbm, agg_hbm, deg_hbm,
             dst_v, src_v, gp_v, gq_v, acc_sh, sem1, sem2,
             ones_v, dacc_sh) = refs
        else:
            (p_hbm, q_hbm, dst_hbm, src_hbm, z_hbm, agg_hbm,
             dst_v, src_v, gp_v, gq_v, acc_sh, sem1, sem2) = refs
        cid = lax.axis_index("c")
        sid = lax.axis_index("s")
        row0 = sid * RPS

        pltpu.sync_copy(z_hbm.at[pl.ds(row0, RPS), :],
                        acc_sh.at[pl.ds(row0, RPS), :])
        pltpu.sync_copy(dst_hbm.at[sid], dst_v)
        pltpu.sync_copy(src_hbm.at[sid], src_v)
        if with_deg:
            pltpu.sync_copy(z_hbm.at[pl.ds(row0, RPS), :DEG_W],
                            dacc_sh.at[pl.ds(row0, RPS), :])

            @pl.when(cid == 0)
            def _():
                @pl.loop(0, CHUNK)
                def _(r):
                    ones_v[r, :] = jnp.full((DEG_W,), 1.0, jnp.float32)

        plsc.subcore_barrier()

        @pl.loop(0, NCHUNKS)
        def _(ch):
            di = dst_v.at[ch]
            si = src_v.at[ch]
            cp1 = pltpu.async_copy(p_hbm.at[cid].at[di], gp_v, sem1)
            cp2 = pltpu.async_copy(q_hbm.at[cid].at[si], gq_v, sem2)
            cp1.wait()
            cp2.wait()

            @pl.loop(0, CHUNK)
            def _(r):
                @pl.loop(0, hh // LANES)
                def _(c):
                    sl = pl.ds(c * LANES, LANES)
                    gp_v[r, sl] = jnp.maximum(gp_v[r, sl] + gq_v[r, sl], 0.0)

            pltpu.sync_copy(gp_v, acc_sh.at[di], add=True)
            if with_deg:
                @pl.when(cid == 0)
                def _():
                    pltpu.sync_copy(ones_v, dacc_sh.at[di], add=True)

        plsc.subcore_barrier()

        pltpu.sync_copy(acc_sh.at[pl.ds(row0, RPS), :],
                        agg_hbm.at[cid].at[pl.ds(row0, RPS), :])
        if with_deg:
            @pl.when(cid == 0)
            def _():
                pltpu.sync_copy(dacc_sh.at[pl.ds(row0, RPS), :],
                                deg_hbm.at[pl.ds(row0, RPS), :])

    return k(p, q, dst3, src3, zeros)


def _head_call(h, deg, wf1, bf1_2d, wf2_row, bf2_2d):
    bn = 1000

    def body(h_ref, deg_ref, w1_ref, b1_ref, w2_ref, b2_ref, o_ref):
        xb = h_ref[...] * (1.0 / jnp.maximum(deg_ref[...][:, :1], 1.0))
        a = jnp.maximum(
            jnp.dot(xb, w1_ref[...], preferred_element_type=jnp.float32)
            + b1_ref[...],
            0.0,
        )
        z = jnp.sum(a * w2_ref[...], axis=1, keepdims=True) + b2_ref[0, 0]
        o_ref[...] = 1.0 / (1.0 + jnp.exp(-z))

    return pl.pallas_call(
        body,
        grid=(N // bn,),
        in_specs=[
            pl.BlockSpec((bn, H3), lambda i: (i, 0)),
            pl.BlockSpec((bn, DEG_W), lambda i: (i, 0)),
            pl.BlockSpec((H3, H_FC), lambda i: (0, 0)),
            pl.BlockSpec((1, H_FC), lambda i: (0, 0)),
            pl.BlockSpec((1, H_FC), lambda i: (0, 0)),
            pl.BlockSpec((1, 1), lambda i: (0, 0)),
        ],
        out_specs=pl.BlockSpec((bn, 1), lambda i: (i, 0)),
        out_shape=jax.ShapeDtypeStruct((N, 1), jnp.float32),
    )(h, deg, wf1, bf1_2d, wf2_row, bf2_2d)


def _merge(agg):
    return jnp.transpose(agg, (1, 0, 2)).reshape(N, -1)


def kernel(X, edge_index, W1, b1, W2, b2, W3, b3, Wf1, bf1, Wf2, bf2):
    ei = edge_index.astype(jnp.int32)
    src3 = ei[0].reshape(NS, NCHUNKS, CHUNK)
    dst3 = ei[1].reshape(NS, NCHUNKS, CHUNK)
    z128 = jnp.zeros((N, 128), jnp.float32)

    p, q = _pq_call(X, W1, b1.reshape(1, -1), None, 128, H1)
    agg, deg = _edge_call(p, q, dst3, src3, z128[:, : H1 // 2], True)

    p, q = _pq_call(_merge(agg), W2, b2.reshape(1, -1), deg, H1, H2)
    agg = _edge_call(p, q, dst3, src3, z128[:, : H2 // 2], False)

    p, q = _pq_call(_merge(agg), W3, b3.reshape(1, -1), deg, H2, H3)
    agg = _edge_call(p, q, dst3, src3, z128, False)

    out = _head_call(
        _merge(agg), deg, Wf1, bf1.reshape(1, -1),
        Wf2.reshape(1, -1), bf2.reshape(1, 1),
    )
    return out[:, 0]


# speedup vs baseline: 4.1870x; 4.1870x over previous
import functools

import jax
import jax.numpy as jnp
from jax import lax
from jax.experimental import pallas as pl
from jax.experimental.pallas import tpu as pltpu
from jax.experimental.pallas import tpu_sc as plsc

N = 10000
NPAD = 10240
E = 320000
NC = 2
NS = 16
LANES = 16
CHUNK = 80
RPS = NPAD // NS
DEG_W = 16
ACC_W = 128

H1, H2, H3, H_FC = 64, 128, 256, 256


def _table_call(x_parts, w, b2d, deg_parts, h_in, h_out):
    bn = 1024
    use_deg = deg_parts is not None
    parts = x_parts.ndim == 3

    def body(*refs):
        if use_deg:
            x_ref, w_ref, b_ref, deg_ref, t_ref = refs
        else:
            x_ref, w_ref, b_ref, t_ref = refs
        xb = (x_ref[0] + x_ref[1]) if parts else x_ref[...]
        if use_deg:
            d = deg_ref[0][:, :1] + deg_ref[1][:, :1]
            xb = xb * (1.0 / jnp.maximum(d, 1.0))
        wa = w_ref[:h_in, :]
        wb = w_ref[h_in:, :]
        wcat = jnp.concatenate([wa - wb, wb], axis=1)
        t_ref[...] = (
            jnp.dot(xb, wcat, preferred_element_type=jnp.float32)
            + b_ref[...]
        )

    xspec = (
        pl.BlockSpec((NC, bn, h_in), lambda i: (0, i, 0))
        if parts
        else pl.BlockSpec((bn, h_in), lambda i: (i, 0))
    )
    in_specs = [
        xspec,
        pl.BlockSpec((2 * h_in, h_out), lambda i: (0, 0)),
        pl.BlockSpec((1, 2 * h_out), lambda i: (0, 0)),
    ]
    args = [x_parts, w, b2d]
    if use_deg:
        in_specs.append(pl.BlockSpec((NC, bn, DEG_W), lambda i: (0, i, 0)))
        args.append(deg_parts)
    return pl.pallas_call(
        body,
        grid=(NPAD // bn,),
        in_specs=in_specs,
        out_specs=pl.BlockSpec((bn, 2 * h_out), lambda i: (i, 0)),
        out_shape=jax.ShapeDtypeStruct((NPAD, 2 * h_out), jnp.float32),
    )(*args)


def _edge_combined_call(t, dst4, src4, zeros, h, chunk, iblk, nblk):
    mesh = plsc.VectorSubcoreMesh(core_axis_name="c", subcore_axis_name="s")
    use_m = h == ACC_W

    scratch = [
        pltpu.VMEM((iblk, chunk), jnp.int32),
        pltpu.VMEM((iblk, chunk), jnp.int32),
        pltpu.VMEM((chunk, 2 * h), jnp.float32),
        pltpu.VMEM((chunk, 2 * h), jnp.float32),
        pltpu.VMEM_SHARED((NPAD, ACC_W), jnp.float32),
        pltpu.SemaphoreType.DMA,
        pltpu.SemaphoreType.DMA,
    ]
    if use_m:
        scratch.insert(4, pltpu.VMEM((chunk, ACC_W), jnp.float32))

    @functools.partial(
        pl.kernel,
        out_type=jax.ShapeDtypeStruct((NC, NPAD, ACC_W), jnp.float32),
        mesh=mesh,
        scratch_types=scratch,
    )
    def k(*refs):
        if use_m:
            (t_hbm, dst_hbm, src_hbm, z_hbm, agg_hbm,
             dst_v, src_v, gd_v, gs_v, m_v, acc_sh, sem1, sem2) = refs
        else:
            (t_hbm, dst_hbm, src_hbm, z_hbm, agg_hbm,
             dst_v, src_v, gd_v, gs_v, acc_sh, sem1, sem2) = refs
        cid = lax.axis_index("c")
        sid = lax.axis_index("s")
        wid = cid * NS + sid
        row0 = sid * RPS

        pltpu.sync_copy(z_hbm.at[pl.ds(row0, RPS), :],
                        acc_sh.at[pl.ds(row0, RPS), :])
        plsc.subcore_barrier()

        @pl.loop(0, nblk)
        def _(blk):
            pltpu.sync_copy(dst_hbm.at[wid].at[blk], dst_v)
            pltpu.sync_copy(src_hbm.at[wid].at[blk], src_v)

            @pl.loop(0, iblk)
            def _(ch):
                di = dst_v.at[ch]
                si = src_v.at[ch]
                cp1 = pltpu.async_copy(t_hbm.at[di], gd_v, sem1)
                cp2 = pltpu.async_copy(t_hbm.at[si], gs_v, sem2)
                cp1.wait()
                cp2.wait()

                if use_m:
                    @pl.loop(0, chunk)
                    def _(r):
                        @pl.loop(0, h // LANES)
                        def _(c):
                            sl = pl.ds(c * LANES, LANES)
                            m_v[r, sl] = jnp.maximum(
                                gd_v[r, sl]
                                + gs_v[r, pl.ds(h + c * LANES, LANES)],
                                0.0,
                            )
                    pltpu.sync_copy(m_v, acc_sh.at[di], add=True)
                else:
                    @pl.loop(0, chunk)
                    def _(r):
                        @pl.loop(0, h // LANES)
                        def _(c):
                            sl = pl.ds(c * LANES, LANES)
                            gd_v[r, sl] = jnp.maximum(
                                gd_v[r, sl]
                                + gs_v[r, pl.ds(h + c * LANES, LANES)],
                                0.0,
                            )
                        gd_v[r, pl.ds(h, LANES)] = jnp.full(
                            (LANES,), 1.0, jnp.float32)

                        @pl.loop(h // LANES + 1, ACC_W // LANES)
                        def _(c):
                            gd_v[r, pl.ds(c * LANES, LANES)] = jnp.zeros(
                                (LANES,), jnp.float32)
                    pltpu.sync_copy(gd_v, acc_sh.at[di], add=True)

        plsc.subcore_barrier()

        pltpu.sync_copy(acc_sh.at[pl.ds(row0, RPS), :],
                        agg_hbm.at[cid].at[pl.ds(row0, RPS), :])

    return k(t, dst4, src4, zeros)


def _edge_split_call(p, q, dst3, src3, zeros):
    hh = p.shape[2]
    nblk, iblk, chunk = dst3.shape[1], dst3.shape[2], dst3.shape[3]
    mesh = plsc.VectorSubcoreMesh(core_axis_name="c", subcore_axis_name="s")

    @functools.partial(
        pl.kernel,
        out_type=jax.ShapeDtypeStruct((NC, NPAD, hh), jnp.float32),
        mesh=mesh,
        scratch_types=[
            pltpu.VMEM((iblk, chunk), jnp.int32),
            pltpu.VMEM((iblk, chunk), jnp.int32),
            pltpu.VMEM((chunk, hh), jnp.float32),
            pltpu.VMEM((chunk, hh), jnp.float32),
            pltpu.VMEM_SHARED((NPAD, hh), jnp.float32),
            pltpu.SemaphoreType.DMA,
            pltpu.SemaphoreType.DMA,
        ],
    )
    def k(p_hbm, q_hbm, dst_hbm, src_hbm, z_hbm, agg_hbm,
          dst_v, src_v, gp_v, gq_v, acc_sh, sem1, sem2):
        cid = lax.axis_index("c")
        sid = lax.axis_index("s")
        row0 = sid * RPS

        pltpu.sync_copy(z_hbm.at[pl.ds(row0, RPS), :],
                        acc_sh.at[pl.ds(row0, RPS), :])
        plsc.subcore_barrier()

        @pl.loop(0, nblk)
        def _(blk):
            pltpu.sync_copy(dst_hbm.at[sid].at[blk], dst_v)
            pltpu.sync_copy(src_hbm.at[sid].at[blk], src_v)

            @pl.loop(0, iblk)
            def _(ch):
                di = dst_v.at[ch]
                si = src_v.at[ch]
                cp1 = pltpu.async_copy(p_hbm.at[cid].at[di], gp_v, sem1)
                cp2 = pltpu.async_copy(q_hbm.at[cid].at[si], gq_v, sem2)
                cp1.wait()
                cp2.wait()

                @pl.loop(0, chunk)
                def _(r):
                    @pl.loop(0, hh // LANES)
                    def _(c):
                        sl = pl.ds(c * LANES, LANES)
                        gp_v[r, sl] = jnp.maximum(
                            gp_v[r, sl] + gq_v[r, sl], 0.0)

                pltpu.sync_copy(gp_v, acc_sh.at[di], add=True)

        plsc.subcore_barrier()

        pltpu.sync_copy(acc_sh.at[pl.ds(row0, RPS), :],
                        agg_hbm.at[cid].at[pl.ds(row0, RPS), :])

    return k(p, q, dst3, src3, zeros)


def _head_call(h_parts, deg_parts, wf1, bf1_2d, wf2_row, bf2_2d):
    bn = 1024

    def body(h_ref, deg_ref, w1_ref, b1_ref, w2_ref, b2_ref, o_ref):
        xb = jnp.concatenate([h_ref[0], h_ref[1]], axis=1)
        d = deg_ref[0][:, :1] + deg_ref[1][:, :1]
        xb = xb * (1.0 / jnp.maximum(d, 1.0))
        a = jnp.maximum(
            jnp.dot(xb, w1_ref[...], preferred_element_type=jnp.float32)
            + b1_ref[...],
            0.0,
        )
        z = jnp.sum(a * w2_ref[...], axis=1, keepdims=True) + b2_ref[0, 0]
        o_ref[...] = 1.0 / (1.0 + jnp.exp(-z))

    return pl.pallas_call(
        body,
        grid=(NPAD // bn,),
        in_specs=[
            pl.BlockSpec((NC, bn, H3 // 2), lambda i: (0, i, 0)),
            pl.BlockSpec((NC, bn, DEG_W), lambda i: (0, i, 0)),
            pl.BlockSpec((H3, H_FC), lambda i: (0, 0)),
            pl.BlockSpec((1, H_FC), lambda i: (0, 0)),
            pl.BlockSpec((1, H_FC), lambda i: (0, 0)),
            pl.BlockSpec((1, 1), lambda i: (0, 0)),
        ],
        out_specs=pl.BlockSpec((bn, 1), lambda i: (i, 0)),
        out_shape=jax.ShapeDtypeStruct((NPAD, 1), jnp.float32),
    )(h_parts, deg_parts, wf1, bf1_2d, wf2_row, bf2_2d)


def _split(t):
    hh = t.shape[1] // 2
    return jnp.transpose(t.reshape(NPAD, NC, hh), (1, 0, 2))


def _bcat(b, h):
    return jnp.concatenate([b, jnp.zeros_like(b)]).reshape(1, 2 * h)


def kernel(X, edge_index, W1, b1, W2, b2, W3, b3, Wf1, bf1, Wf2, bf2):
    ei = edge_index.astype(jnp.int32)
    d4a = ei[1].reshape(NC * NS, 5, 25, 80)
    s4a = ei[0].reshape(NC * NS, 5, 25, 80)
    d4b = ei[1].reshape(NC * NS, 10, 25, 40)
    s4b = ei[0].reshape(NC * NS, 10, 25, 40)
    d3 = ei[1].reshape(NS, 10, 25, 80)
    s3 = ei[0].reshape(NS, 10, 25, 80)
    z128 = jnp.zeros((NPAD, ACC_W), jnp.float32)
    xpad = jnp.pad(X, ((0, NPAD - N), (0, 0)))

    t = _table_call(xpad, W1, _bcat(b1, H1), None, 128, H1)
    agg = _edge_combined_call(t, d4a, s4a, z128, H1, 80, 25, 5)
    deg = agg[:, :, H1:H1 + DEG_W]

    t = _table_call(agg[:, :, :H1], W2, _bcat(b2, H2), deg, H1, H2)
    agg = _edge_combined_call(t, d4b, s4b, z128, H2, 40, 25, 10)

    t = _table_call(agg, W3, _bcat(b3, H3), deg, H2, H3)
    agg = _edge_split_call(_split(t[:, :H3]), _split(t[:, H3:]), d3, s3,
                           z128)

    out = _head_call(
        agg, deg, Wf1, bf1.reshape(1, -1),
        Wf2.reshape(1, -1), bf2.reshape(1, 1),
    )
    return out[:N, 0]


# speedup vs baseline: 4.3967x; 1.0501x over previous
import functools

import jax
import jax.numpy as jnp
from jax import lax
from jax.experimental import pallas as pl
from jax.experimental.pallas import tpu as pltpu
from jax.experimental.pallas import tpu_sc as plsc

N = 10000
NPAD = 10240
E = 320000
NC = 2
NS = 16
LANES = 16
CHUNK = 80
RPS = NPAD // NS
DEG_W = 16
ACC_W = 128

H1, H2, H3, H_FC = 64, 128, 256, 256


def _table_call(x_parts, w, b2d, deg_parts, h_in, h_out):
    bn = 1024
    use_deg = deg_parts is not None
    parts = x_parts.ndim == 3

    def body(*refs):
        if use_deg:
            x_ref, w_ref, b_ref, deg_ref, t_ref = refs
        else:
            x_ref, w_ref, b_ref, t_ref = refs
        xb = (x_ref[0] + x_ref[1]) if parts else x_ref[...]
        if use_deg:
            d = deg_ref[0][:, :1] + deg_ref[1][:, :1]
            xb = xb * (1.0 / jnp.maximum(d, 1.0))
        wa = w_ref[:h_in, :]
        wb = w_ref[h_in:, :]
        wcat = jnp.concatenate([wa - wb, wb], axis=1)
        t_ref[...] = (
            jnp.dot(xb, wcat, preferred_element_type=jnp.float32)
            + b_ref[...]
        )

    xspec = (
        pl.BlockSpec((NC, bn, h_in), lambda i: (0, i, 0))
        if parts
        else pl.BlockSpec((bn, h_in), lambda i: (i, 0))
    )
    in_specs = [
        xspec,
        pl.BlockSpec((2 * h_in, h_out), lambda i: (0, 0)),
        pl.BlockSpec((1, 2 * h_out), lambda i: (0, 0)),
    ]
    args = [x_parts, w, b2d]
    if use_deg:
        in_specs.append(pl.BlockSpec((NC, bn, DEG_W), lambda i: (0, i, 0)))
        args.append(deg_parts)
    return pl.pallas_call(
        body,
        grid=(NPAD // bn,),
        in_specs=in_specs,
        out_specs=pl.BlockSpec((bn, 2 * h_out), lambda i: (i, 0)),
        out_shape=jax.ShapeDtypeStruct((NPAD, 2 * h_out), jnp.float32),
    )(*args)


def _edge_combined_call(t, dst4, src4, zeros, h, chunk, iblk, nblk,
                        feature_split, with_ones):
    mesh = plsc.VectorSubcoreMesh(core_axis_name="c", subcore_axis_name="s")

    scratch = [
        pltpu.VMEM((iblk, chunk), jnp.int32),
        pltpu.VMEM((iblk, chunk), jnp.int32),
        pltpu.VMEM((chunk, 2 * h), jnp.float32),
        pltpu.VMEM((chunk, 2 * h), jnp.float32),
        pltpu.VMEM((chunk, 2 * h), jnp.float32),
        pltpu.VMEM((chunk, 2 * h), jnp.float32),
        pltpu.VMEM_SHARED((NPAD, ACC_W), jnp.float32),
        pltpu.SemaphoreType.DMA,
        pltpu.SemaphoreType.DMA,
        pltpu.SemaphoreType.DMA,
        pltpu.SemaphoreType.DMA,
    ]

    @functools.partial(
        pl.kernel,
        out_type=jax.ShapeDtypeStruct((NC, NPAD, ACC_W), jnp.float32),
        mesh=mesh,
        scratch_types=scratch,
    )
    def k(t_hbm, dst_hbm, src_hbm, z_hbm, agg_hbm,
          dst_v, src_v, gd_a, gs_a, gd_b, gs_b,
          acc_sh, sem_ad, sem_as, sem_bd, sem_bs):
        cid = lax.axis_index("c")
        sid = lax.axis_index("s")
        wid = sid if feature_split else cid * NS + sid
        tbl = t_hbm.at[cid] if feature_split else t_hbm
        row0 = sid * RPS

        pltpu.sync_copy(z_hbm.at[pl.ds(row0, RPS), :],
                        acc_sh.at[pl.ds(row0, RPS), :])
        plsc.subcore_barrier()

        def issue(ch, gd, gs, semd, sems):
            return (pltpu.async_copy(tbl.at[dst_v.at[ch]], gd, semd),
                    pltpu.async_copy(tbl.at[src_v.at[ch]], gs, sems))

        def combine_scatter(ch, gd, gs):
            @pl.loop(0, chunk)
            def _(r):
                for c in range(h // LANES):
                    sl = pl.ds(c * LANES, LANES)
                    gd[r, sl] = jnp.maximum(
                        gd[r, sl] + gs[r, pl.ds(h + c * LANES, LANES)],
                        0.0,
                    )
                if with_ones:
                    gd[r, pl.ds(h, LANES)] = jnp.full(
                        (LANES,), 1.0, jnp.float32)
            pltpu.sync_copy(gd, acc_sh.at[dst_v.at[ch]], add=True)

        @pl.loop(0, nblk)
        def _(blk):
            pltpu.sync_copy(dst_hbm.at[wid].at[blk], dst_v)
            pltpu.sync_copy(src_hbm.at[wid].at[blk], src_v)

            @pl.loop(0, iblk // 2)
            def _(i):
                cpa = issue(2 * i, gd_a, gs_a, sem_ad, sem_as)
                cpb = issue(2 * i + 1, gd_b, gs_b, sem_bd, sem_bs)
                cpa[0].wait()
                cpa[1].wait()
                combine_scatter(2 * i, gd_a, gs_a)
                cpb[0].wait()
                cpb[1].wait()
                combine_scatter(2 * i + 1, gd_b, gs_b)

        plsc.subcore_barrier()

        pltpu.sync_copy(acc_sh.at[pl.ds(row0, RPS), :],
                        agg_hbm.at[cid].at[pl.ds(row0, RPS), :])

    return k(t, dst4, src4, zeros)


def _edge_split_call(p, q, dst3, src3, zeros):
    hh = p.shape[2]
    nblk, iblk, chunk = dst3.shape[1], dst3.shape[2], dst3.shape[3]
    mesh = plsc.VectorSubcoreMesh(core_axis_name="c", subcore_axis_name="s")

    @functools.partial(
        pl.kernel,
        out_type=jax.ShapeDtypeStruct((NC, NPAD, hh), jnp.float32),
        mesh=mesh,
        scratch_types=[
            pltpu.VMEM((iblk, chunk), jnp.int32),
            pltpu.VMEM((iblk, chunk), jnp.int32),
            pltpu.VMEM((chunk, hh), jnp.float32),
            pltpu.VMEM((chunk, hh), jnp.float32),
            pltpu.VMEM((chunk, hh), jnp.float32),
            pltpu.VMEM((chunk, hh), jnp.float32),
            pltpu.VMEM_SHARED((NPAD, hh), jnp.float32),
            pltpu.SemaphoreType.DMA,
            pltpu.SemaphoreType.DMA,
            pltpu.SemaphoreType.DMA,
            pltpu.SemaphoreType.DMA,
        ],
    )
    def k(p_hbm, q_hbm, dst_hbm, src_hbm, z_hbm, agg_hbm,
          dst_v, src_v, gp_a, gq_a, gp_b, gq_b, acc_sh,
          sem_ad, sem_as, sem_bd, sem_bs):
        cid = lax.axis_index("c")
        sid = lax.axis_index("s")
        row0 = sid * RPS

        pltpu.sync_copy(z_hbm.at[pl.ds(row0, RPS), :],
                        acc_sh.at[pl.ds(row0, RPS), :])
        plsc.subcore_barrier()

        def issue(ch, gp, gq, semd, sems):
            return (pltpu.async_copy(p_hbm.at[cid].at[dst_v.at[ch]], gp,
                                     semd),
                    pltpu.async_copy(q_hbm.at[cid].at[src_v.at[ch]], gq,
                                     sems))

        def combine_scatter(ch, gp, gq):
            @pl.loop(0, chunk)
            def _(r):
                for c in range(hh // LANES):
                    sl = pl.ds(c * LANES, LANES)
                    gp[r, sl] = jnp.maximum(gp[r, sl] + gq[r, sl], 0.0)
            pltpu.sync_copy(gp, acc_sh.at[dst_v.at[ch]], add=True)

        @pl.loop(0, nblk)
        def _(blk):
            pltpu.sync_copy(dst_hbm.at[sid].at[blk], dst_v)
            pltpu.sync_copy(src_hbm.at[sid].at[blk], src_v)
            @pl.loop(0, iblk // 2)
            def _(i):
                cpa = issue(2 * i, gp_a, gq_a, sem_ad, sem_as)
                cpb = issue(2 * i + 1, gp_b, gq_b, sem_bd, sem_bs)
                cpa[0].wait()
                cpa[1].wait()
                combine_scatter(2 * i, gp_a, gq_a)
                cpb[0].wait()
                cpb[1].wait()
                combine_scatter(2 * i + 1, gp_b, gq_b)

        plsc.subcore_barrier()

        pltpu.sync_copy(acc_sh.at[pl.ds(row0, RPS), :],
                        agg_hbm.at[cid].at[pl.ds(row0, RPS), :])

    return k(p, q, dst3, src3, zeros)


def _head_call(h_parts, deg_parts, wf1, bf1_2d, wf2_row, bf2_2d):
    bn = 1024

    def body(h_ref, deg_ref, w1_ref, b1_ref, w2_ref, b2_ref, o_ref):
        xb = jnp.concatenate([h_ref[0], h_ref[1]], axis=1)
        d = deg_ref[0][:, :1] + deg_ref[1][:, :1]
        xb = xb * (1.0 / jnp.maximum(d, 1.0))
        a = jnp.maximum(
            jnp.dot(xb, w1_ref[...], preferred_element_type=jnp.float32)
            + b1_ref[...],
            0.0,
        )
        z = jnp.sum(a * w2_ref[...], axis=1, keepdims=True) + b2_ref[0, 0]
        o_ref[...] = 1.0 / (1.0 + jnp.exp(-z))

    return pl.pallas_call(
        body,
        grid=(NPAD // bn,),
        in_specs=[
            pl.BlockSpec((NC, bn, H3 // 2), lambda i: (0, i, 0)),
            pl.BlockSpec((NC, bn, DEG_W), lambda i: (0, i, 0)),
            pl.BlockSpec((H3, H_FC), lambda i: (0, 0)),
            pl.BlockSpec((1, H_FC), lambda i: (0, 0)),
            pl.BlockSpec((1, H_FC), lambda i: (0, 0)),
            pl.BlockSpec((1, 1), lambda i: (0, 0)),
        ],
        out_specs=pl.BlockSpec((bn, 1), lambda i: (i, 0)),
        out_shape=jax.ShapeDtypeStruct((NPAD, 1), jnp.float32),
    )(h_parts, deg_parts, wf1, bf1_2d, wf2_row, bf2_2d)


def _split(t):
    hh = t.shape[1] // 2
    return jnp.transpose(t.reshape(NPAD, NC, hh), (1, 0, 2))


def _bcat(b, h):
    return jnp.concatenate([b, jnp.zeros_like(b)]).reshape(1, 2 * h)


def kernel(X, edge_index, W1, b1, W2, b2, W3, b3, Wf1, bf1, Wf2, bf2):
    ei = edge_index.astype(jnp.int32)
    d32 = ei[1].reshape(NC * NS, 25, 10, 40)
    s32 = ei[0].reshape(NC * NS, 25, 10, 40)
    d16 = ei[1].reshape(NS, 50, 10, 40)
    s16 = ei[0].reshape(NS, 50, 10, 40)
    z128 = jnp.zeros((NPAD, ACC_W), jnp.float32)
    xpad = jnp.pad(X, ((0, NPAD - N), (0, 0)))

    t = _table_call(xpad, W1, _bcat(b1, H1), None, 128, H1)
    agg = _edge_combined_call(t, d32, s32, z128, H1, 40, 10, 25,
                              feature_split=False, with_ones=True)
    deg = agg[:, :, H1:H1 + DEG_W]

    t = _table_call(agg[:, :, :H1], W2, _bcat(b2, H2), deg, H1, H2)
    t2 = jnp.stack([
        jnp.concatenate([t[:, :H1], t[:, H2:H2 + H1]], axis=1),
        jnp.concatenate([t[:, H1:H2], t[:, H2 + H1:]], axis=1),
    ])
    agg = _edge_combined_call(t2, d16, s16, z128, H1, 40, 10, 50,
                              feature_split=True, with_ones=False)
    x3 = jnp.concatenate([agg[0, :, :H1], agg[1, :, :H1]], axis=1)

    t = _table_call(x3, W3, _bcat(b3, H3), deg, H2, H3)
    agg = _edge_split_call(_split(t[:, :H3]), _split(t[:, H3:]), d16, s16,
                           z128)

    out = _head_call(
        agg, deg, Wf1, bf1.reshape(1, -1),
        Wf2.reshape(1, -1), bf2.reshape(1, 1),
    )
    return out[:N, 0]


# speedup vs baseline: 5.0638x; 1.1517x over previous
import functools

import jax
import jax.numpy as jnp
from jax import lax
from jax.experimental import pallas as pl
from jax.experimental.pallas import tpu as pltpu
from jax.experimental.pallas import tpu_sc as plsc

N = 10000
NPAD = 10240
E = 320000
NC = 2
NS = 16
LANES = 16
CHUNK = 80
RPS = NPAD // NS
DEG_W = 16
ACC_W = 128

H1, H2, H3, H_FC = 64, 128, 256, 256


def _table_call(x_parts, w, b2d, deg_parts, h_in, h_out):
    bn = 1024
    use_deg = deg_parts is not None
    parts = x_parts.ndim == 3

    def body(*refs):
        if use_deg:
            x_ref, w_ref, b_ref, deg_ref, t_ref = refs
        else:
            x_ref, w_ref, b_ref, t_ref = refs
        xb = (x_ref[0] + x_ref[1]) if parts else x_ref[...]
        if use_deg:
            d = deg_ref[0][:, :1] + deg_ref[1][:, :1]
            xb = xb * (1.0 / jnp.maximum(d, 1.0))
        wa = w_ref[:h_in, :]
        wb = w_ref[h_in:, :]
        wcat = jnp.concatenate([wa - wb, wb], axis=1)
        t_ref[...] = (
            jnp.dot(xb, wcat, preferred_element_type=jnp.float32)
            + b_ref[...]
        )

    xspec = (
        pl.BlockSpec((NC, bn, h_in), lambda i: (0, i, 0))
        if parts
        else pl.BlockSpec((bn, h_in), lambda i: (i, 0))
    )
    in_specs = [
        xspec,
        pl.BlockSpec((2 * h_in, h_out), lambda i: (0, 0)),
        pl.BlockSpec((1, 2 * h_out), lambda i: (0, 0)),
    ]
    args = [x_parts, w, b2d]
    if use_deg:
        in_specs.append(pl.BlockSpec((NC, bn, DEG_W), lambda i: (0, i, 0)))
        args.append(deg_parts)
    return pl.pallas_call(
        body,
        grid=(NPAD // bn,),
        in_specs=in_specs,
        out_specs=pl.BlockSpec((bn, 2 * h_out), lambda i: (i, 0)),
        out_shape=jax.ShapeDtypeStruct((NPAD, 2 * h_out), jnp.float32),
    )(*args)


def _edge_combined_call(t, dst4, src4, zeros, h, chunk, iblk, nblk,
                        feature_split, with_ones):
    mesh = plsc.VectorSubcoreMesh(core_axis_name="c", subcore_axis_name="s")

    scratch = [
        pltpu.VMEM((iblk, chunk), jnp.int32),
        pltpu.VMEM((iblk, chunk), jnp.int32),
        pltpu.VMEM((chunk, 2 * h), jnp.float32),
        pltpu.VMEM((chunk, 2 * h), jnp.float32),
        pltpu.VMEM((chunk, 2 * h), jnp.float32),
        pltpu.VMEM((chunk, 2 * h), jnp.float32),
        pltpu.VMEM_SHARED((NPAD, ACC_W), jnp.float32),
        pltpu.SemaphoreType.DMA,
        pltpu.SemaphoreType.DMA,
        pltpu.SemaphoreType.DMA,
        pltpu.SemaphoreType.DMA,
    ]

    @functools.partial(
        pl.kernel,
        out_type=jax.ShapeDtypeStruct((NC, NPAD, ACC_W), jnp.float32),
        mesh=mesh,
        scratch_types=scratch,
    )
    def k(t_hbm, dst_hbm, src_hbm, z_hbm, agg_hbm,
          dst_v, src_v, gd_a, gs_a, gd_b, gs_b,
          acc_sh, sem_ad, sem_as, sem_bd, sem_bs):
        cid = lax.axis_index("c")
        sid = lax.axis_index("s")
        wid = sid if feature_split else cid * NS + sid
        tbl = t_hbm.at[cid] if feature_split else t_hbm
        row0 = sid * RPS

        pltpu.sync_copy(z_hbm.at[pl.ds(row0, RPS), :],
                        acc_sh.at[pl.ds(row0, RPS), :])
        plsc.subcore_barrier()

        def issue(ch, gd, gs, semd, sems):
            return (pltpu.async_copy(tbl.at[dst_v.at[ch]], gd, semd),
                    pltpu.async_copy(tbl.at[src_v.at[ch]], gs, sems))

        def combine_scatter(ch, gd, gs):
            @pl.loop(0, chunk)
            def _(r):
                for c in range(h // LANES):
                    sl = pl.ds(c * LANES, LANES)
                    gd[r, sl] = jnp.maximum(
                        gd[r, sl] + gs[r, pl.ds(h + c * LANES, LANES)],
                        0.0,
                    )
                if with_ones:
                    gd[r, pl.ds(h, LANES)] = jnp.full(
                        (LANES,), 1.0, jnp.float32)
            pltpu.sync_copy(gd, acc_sh.at[dst_v.at[ch]], add=True)

        @pl.loop(0, nblk)
        def _(blk):
            pltpu.sync_copy(dst_hbm.at[wid].at[blk], dst_v)
            pltpu.sync_copy(src_hbm.at[wid].at[blk], src_v)

            @pl.loop(0, iblk // 2)
            def _(i):
                cpa = issue(2 * i, gd_a, gs_a, sem_ad, sem_as)
                cpb = issue(2 * i + 1, gd_b, gs_b, sem_bd, sem_bs)
                cpa[0].wait()
                cpa[1].wait()
                combine_scatter(2 * i, gd_a, gs_a)
                cpb[0].wait()
                cpb[1].wait()
                combine_scatter(2 * i + 1, gd_b, gs_b)

            if iblk % 2:
                cpa = issue(iblk - 1, gd_a, gs_a, sem_ad, sem_as)
                cpa[0].wait()
                cpa[1].wait()
                combine_scatter(iblk - 1, gd_a, gs_a)

        plsc.subcore_barrier()

        pltpu.sync_copy(acc_sh.at[pl.ds(row0, RPS), :],
                        agg_hbm.at[cid].at[pl.ds(row0, RPS), :])

    return k(t, dst4, src4, zeros)


def _edge_split_call(p, q, dst3, src3, zeros):
    hh = p.shape[2]
    nblk, iblk, chunk = dst3.shape[1], dst3.shape[2], dst3.shape[3]
    mesh = plsc.VectorSubcoreMesh(core_axis_name="c", subcore_axis_name="s")

    @functools.partial(
        pl.kernel,
        out_type=jax.ShapeDtypeStruct((NC, NPAD, hh), jnp.float32),
        mesh=mesh,
        scratch_types=[
            pltpu.VMEM((iblk, chunk), jnp.int32),
            pltpu.VMEM((iblk, chunk), jnp.int32),
            pltpu.VMEM((chunk, hh), jnp.float32),
            pltpu.VMEM((chunk, hh), jnp.float32),
            pltpu.VMEM((chunk, hh), jnp.float32),
            pltpu.VMEM((chunk, hh), jnp.float32),
            pltpu.VMEM_SHARED((NPAD, hh), jnp.float32),
            pltpu.SemaphoreType.DMA,
            pltpu.SemaphoreType.DMA,
            pltpu.SemaphoreType.DMA,
            pltpu.SemaphoreType.DMA,
        ],
    )
    def k(p_hbm, q_hbm, dst_hbm, src_hbm, z_hbm, agg_hbm,
          dst_v, src_v, gp_a, gq_a, gp_b, gq_b, acc_sh,
          sem_ad, sem_as, sem_bd, sem_bs):
        cid = lax.axis_index("c")
        sid = lax.axis_index("s")
        row0 = sid * RPS

        pltpu.sync_copy(z_hbm.at[pl.ds(row0, RPS), :],
                        acc_sh.at[pl.ds(row0, RPS), :])
        plsc.subcore_barrier()

        def issue(ch, gp, gq, semd, sems):
            return (pltpu.async_copy(p_hbm.at[cid].at[dst_v.at[ch]], gp,
                                     semd),
                    pltpu.async_copy(q_hbm.at[cid].at[src_v.at[ch]], gq,
                                     sems))

        def combine_scatter(ch, gp, gq):
            @pl.loop(0, chunk)
            def _(r):
                for c in range(hh // LANES):
                    sl = pl.ds(c * LANES, LANES)
                    gp[r, sl] = jnp.maximum(gp[r, sl] + gq[r, sl], 0.0)
            pltpu.sync_copy(gp, acc_sh.at[dst_v.at[ch]], add=True)

        @pl.loop(0, nblk)
        def _(blk):
            pltpu.sync_copy(dst_hbm.at[sid].at[blk], dst_v)
            pltpu.sync_copy(src_hbm.at[sid].at[blk], src_v)
            @pl.loop(0, iblk // 2)
            def _(i):
                cpa = issue(2 * i, gp_a, gq_a, sem_ad, sem_as)
                cpb = issue(2 * i + 1, gp_b, gq_b, sem_bd, sem_bs)
                cpa[0].wait()
                cpa[1].wait()
                combine_scatter(2 * i, gp_a, gq_a)
                cpb[0].wait()
                cpb[1].wait()
                combine_scatter(2 * i + 1, gp_b, gq_b)

        plsc.subcore_barrier()

        pltpu.sync_copy(acc_sh.at[pl.ds(row0, RPS), :],
                        agg_hbm.at[cid].at[pl.ds(row0, RPS), :])

    return k(p, q, dst3, src3, zeros)


def _head_call(h_parts, deg_parts, wf1, bf1_2d, wf2_row, bf2_2d):
    bn = 1024

    def body(h_ref, deg_ref, w1_ref, b1_ref, w2_ref, b2_ref, o_ref):
        xb = jnp.concatenate([h_ref[0], h_ref[1]], axis=1)
        d = deg_ref[0][:, :1] + deg_ref[1][:, :1]
        xb = xb * (1.0 / jnp.maximum(d, 1.0))
        a = jnp.maximum(
            jnp.dot(xb, w1_ref[...], preferred_element_type=jnp.float32)
            + b1_ref[...],
            0.0,
        )
        z = jnp.sum(a * w2_ref[...], axis=1, keepdims=True) + b2_ref[0, 0]
        o_ref[...] = 1.0 / (1.0 + jnp.exp(-z))

    return pl.pallas_call(
        body,
        grid=(NPAD // bn,),
        in_specs=[
            pl.BlockSpec((NC, bn, H3 // 2), lambda i: (0, i, 0)),
            pl.BlockSpec((NC, bn, DEG_W), lambda i: (0, i, 0)),
            pl.BlockSpec((H3, H_FC), lambda i: (0, 0)),
            pl.BlockSpec((1, H_FC), lambda i: (0, 0)),
            pl.BlockSpec((1, H_FC), lambda i: (0, 0)),
            pl.BlockSpec((1, 1), lambda i: (0, 0)),
        ],
        out_specs=pl.BlockSpec((bn, 1), lambda i: (i, 0)),
        out_shape=jax.ShapeDtypeStruct((NPAD, 1), jnp.float32),
    )(h_parts, deg_parts, wf1, bf1_2d, wf2_row, bf2_2d)


def _split(t):
    hh = t.shape[1] // 2
    return jnp.transpose(t.reshape(NPAD, NC, hh), (1, 0, 2))


def _bcat(b, h):
    return jnp.concatenate([b, jnp.zeros_like(b)]).reshape(1, 2 * h)


def kernel(X, edge_index, W1, b1, W2, b2, W3, b3, Wf1, bf1, Wf2, bf2):
    ei = edge_index.astype(jnp.int32)
    d32 = ei[1].reshape(NC * NS, 25, 5, 80)
    s32 = ei[0].reshape(NC * NS, 25, 5, 80)
    d16 = ei[1].reshape(NS, 25, 10, 80)
    s16 = ei[0].reshape(NS, 25, 10, 80)
    z128 = jnp.zeros((NPAD, ACC_W), jnp.float32)
    xpad = jnp.pad(X, ((0, NPAD - N), (0, 0)))

    t = _table_call(xpad, W1, _bcat(b1, H1), None, 128, H1)
    agg = _edge_combined_call(t, d32, s32, z128, H1, 80, 5, 25,
                              feature_split=False, with_ones=True)
    deg = agg[:, :, H1:H1 + DEG_W]

    t = _table_call(agg[:, :, :H1], W2, _bcat(b2, H2), deg, H1, H2)
    t2 = jnp.stack([
        jnp.concatenate([t[:, :H1], t[:, H2:H2 + H1]], axis=1),
        jnp.concatenate([t[:, H1:H2], t[:, H2 + H1:]], axis=1),
    ])
    agg = _edge_combined_call(t2, d16, s16, z128, H1, 80, 10, 25,
                              feature_split=True, with_ones=False)
    x3 = jnp.concatenate([agg[0, :, :H1], agg[1, :, :H1]], axis=1)

    t = _table_call(x3, W3, _bcat(b3, H3), deg, H2, H3)
    agg = _edge_split_call(_split(t[:, :H3]), _split(t[:, H3:]), d16, s16,
                           z128)

    out = _head_call(
        agg, deg, Wf1, bf1.reshape(1, -1),
        Wf2.reshape(1, -1), bf2.reshape(1, 1),
    )
    return out[:N, 0]


# speedup vs baseline: 5.3996x; 1.0663x over previous
import functools

import jax
import jax.numpy as jnp
from jax import lax
from jax.experimental import pallas as pl
from jax.experimental.pallas import tpu as pltpu
from jax.experimental.pallas import tpu_sc as plsc

N = 10000
NPAD = 10240
E = 320000
NC = 2
NS = 16
LANES = 16
RPS = NPAD // NS
DEG_W = 16
ACC_W = 128

H1, H2, H3, H_FC = 64, 128, 256, 256


def _table_call(x_parts, w, b2d, deg_parts, h_in, h_out):
    bn = 1024
    use_deg = deg_parts is not None
    parts = x_parts.ndim == 3

    def body(*refs):
        if use_deg:
            x_ref, w_ref, b_ref, deg_ref, t_ref = refs
        else:
            x_ref, w_ref, b_ref, t_ref = refs
        xb = (x_ref[0] + x_ref[1]) if parts else x_ref[...]
        if use_deg:
            d = deg_ref[0][:, :1] + deg_ref[1][:, :1]
            xb = xb * (1.0 / jnp.maximum(d, 1.0))
        wa = w_ref[:h_in, :]
        wb = w_ref[h_in:, :]
        wcat = jnp.concatenate([wa - wb, wb], axis=1)
        t_ref[...] = (
            jnp.dot(xb, wcat, preferred_element_type=jnp.float32)
            + b_ref[...]
        )

    xspec = (
        pl.BlockSpec((NC, bn, h_in), lambda i: (0, i, 0))
        if parts
        else pl.BlockSpec((bn, h_in), lambda i: (i, 0))
    )
    in_specs = [
        xspec,
        pl.BlockSpec((2 * h_in, h_out), lambda i: (0, 0)),
        pl.BlockSpec((1, 2 * h_out), lambda i: (0, 0)),
    ]
    args = [x_parts, w, b2d]
    if use_deg:
        in_specs.append(pl.BlockSpec((NC, bn, DEG_W), lambda i: (0, i, 0)))
        args.append(deg_parts)
    return pl.pallas_call(
        body,
        grid=(NPAD // bn,),
        in_specs=in_specs,
        out_specs=pl.BlockSpec((bn, 2 * h_out), lambda i: (i, 0)),
        out_shape=jax.ShapeDtypeStruct((NPAD, 2 * h_out), jnp.float32),
    )(*args)


def _edge_combined_call(t, dst4, src4, zeros, h, chunk, iblk, nblk,
                        feature_split, with_ones):
    mesh = plsc.VectorSubcoreMesh(core_axis_name="c", subcore_axis_name="s")

    scratch = [
        pltpu.VMEM((iblk, chunk), jnp.int32),
        pltpu.VMEM((iblk, chunk), jnp.int32),
        pltpu.VMEM((chunk, 2 * h), jnp.float32),
        pltpu.VMEM((chunk, 2 * h), jnp.float32),
        pltpu.VMEM((chunk, 2 * h), jnp.float32),
        pltpu.VMEM((chunk, 2 * h), jnp.float32),
        pltpu.VMEM_SHARED((NPAD, ACC_W), jnp.float32),
        pltpu.SemaphoreType.DMA,
        pltpu.SemaphoreType.DMA,
        pltpu.SemaphoreType.DMA,
        pltpu.SemaphoreType.DMA,
        pltpu.SemaphoreType.DMA,
        pltpu.SemaphoreType.DMA,
    ]

    @functools.partial(
        pl.kernel,
        out_type=jax.ShapeDtypeStruct((NC, NPAD, ACC_W), jnp.float32),
        mesh=mesh,
        scratch_types=scratch,
    )
    def k(t_hbm, dst_hbm, src_hbm, z_hbm, agg_hbm,
          dst_v, src_v, gd_a, gs_a, gd_b, gs_b,
          acc_sh, sem_ad, sem_as, sem_bd, sem_bs, sem_sa, sem_sb):
        cid = lax.axis_index("c")
        sid = lax.axis_index("s")
        wid = sid if feature_split else cid * NS + sid
        tbl = t_hbm.at[cid] if feature_split else t_hbm
        row0 = sid * RPS

        pltpu.sync_copy(z_hbm.at[pl.ds(row0, RPS), :],
                        acc_sh.at[pl.ds(row0, RPS), :])
        plsc.subcore_barrier()

        def issue(ch, gd, gs, semd, sems):
            return (pltpu.async_copy(tbl.at[dst_v.at[ch]], gd, semd),
                    pltpu.async_copy(tbl.at[src_v.at[ch]], gs, sems))

        def combine(gd, gs):
            @pl.loop(0, chunk)
            def _(r):
                for c in range(h // LANES):
                    sl = pl.ds(c * LANES, LANES)
                    gd[r, sl] = jnp.maximum(
                        gd[r, sl] + gs[r, pl.ds(h + c * LANES, LANES)],
                        0.0,
                    )
                if with_ones:
                    gd[r, pl.ds(h, LANES)] = jnp.full(
                        (LANES,), 1.0, jnp.float32)

        def scatter(ch, gd, sem):
            return pltpu.async_copy(gd, acc_sh.at[dst_v.at[ch]], sem,
                                    add=True)

        @pl.loop(0, nblk)
        def _(blk):
            pltpu.sync_copy(dst_hbm.at[wid].at[blk], dst_v)
            pltpu.sync_copy(src_hbm.at[wid].at[blk], src_v)

            @pl.loop(0, iblk // 2)
            def _(i):
                cpa = issue(2 * i, gd_a, gs_a, sem_ad, sem_as)
                cpb = issue(2 * i + 1, gd_b, gs_b, sem_bd, sem_bs)
                cpa[0].wait()
                cpa[1].wait()
                combine(gd_a, gs_a)
                sca = scatter(2 * i, gd_a, sem_sa)
                cpb[0].wait()
                cpb[1].wait()
                combine(gd_b, gs_b)
                scb = scatter(2 * i + 1, gd_b, sem_sb)
                sca.wait()
                scb.wait()

            if iblk % 2:
                cpa = issue(iblk - 1, gd_a, gs_a, sem_ad, sem_as)
                cpa[0].wait()
                cpa[1].wait()
                combine(gd_a, gs_a)
                scatter(iblk - 1, gd_a, sem_sa).wait()

        plsc.subcore_barrier()

        pltpu.sync_copy(acc_sh.at[pl.ds(row0, RPS), :],
                        agg_hbm.at[cid].at[pl.ds(row0, RPS), :])

    return k(t, dst4, src4, zeros)


def _edge_split_call(p, q, dst3, src3, zeros):
    hh = p.shape[2]
    nblk, iblk, chunk = dst3.shape[1], dst3.shape[2], dst3.shape[3]
    mesh = plsc.VectorSubcoreMesh(core_axis_name="c", subcore_axis_name="s")

    @functools.partial(
        pl.kernel,
        out_type=jax.ShapeDtypeStruct((NC, NPAD, hh), jnp.float32),
        mesh=mesh,
        scratch_types=[
            pltpu.VMEM((iblk, chunk), jnp.int32),
            pltpu.VMEM((iblk, chunk), jnp.int32),
            pltpu.VMEM((chunk, hh), jnp.float32),
            pltpu.VMEM((chunk, hh), jnp.float32),
            pltpu.VMEM((chunk, hh), jnp.float32),
            pltpu.VMEM((chunk, hh), jnp.float32),
            pltpu.VMEM_SHARED((NPAD, hh), jnp.float32),
            pltpu.SemaphoreType.DMA,
            pltpu.SemaphoreType.DMA,
            pltpu.SemaphoreType.DMA,
            pltpu.SemaphoreType.DMA,
            pltpu.SemaphoreType.DMA,
            pltpu.SemaphoreType.DMA,
        ],
    )
    def k(p_hbm, q_hbm, dst_hbm, src_hbm, z_hbm, agg_hbm,
          dst_v, src_v, gp_a, gq_a, gp_b, gq_b, acc_sh,
          sem_ad, sem_as, sem_bd, sem_bs, sem_sa, sem_sb):
        cid = lax.axis_index("c")
        sid = lax.axis_index("s")
        row0 = sid * RPS

        pltpu.sync_copy(z_hbm.at[pl.ds(row0, RPS), :],
                        acc_sh.at[pl.ds(row0, RPS), :])
        plsc.subcore_barrier()

        def issue(ch, gp, gq, semd, sems):
            return (pltpu.async_copy(p_hbm.at[cid].at[dst_v.at[ch]], gp,
                                     semd),
                    pltpu.async_copy(q_hbm.at[cid].at[src_v.at[ch]], gq,
                                     sems))

        def combine(gp, gq):
            @pl.loop(0, chunk)
            def _(r):
                for c in range(hh // LANES):
                    sl = pl.ds(c * LANES, LANES)
                    gp[r, sl] = jnp.maximum(gp[r, sl] + gq[r, sl], 0.0)

        def scatter(ch, gp, sem):
            return pltpu.async_copy(gp, acc_sh.at[dst_v.at[ch]], sem,
                                    add=True)

        @pl.loop(0, nblk)
        def _(blk):
            pltpu.sync_copy(dst_hbm.at[sid].at[blk], dst_v)
            pltpu.sync_copy(src_hbm.at[sid].at[blk], src_v)

            @pl.loop(0, iblk // 2)
            def _(i):
                cpa = issue(2 * i, gp_a, gq_a, sem_ad, sem_as)
                cpb = issue(2 * i + 1, gp_b, gq_b, sem_bd, sem_bs)
                cpa[0].wait()
                cpa[1].wait()
                combine(gp_a, gq_a)
                sca = scatter(2 * i, gp_a, sem_sa)
                cpb[0].wait()
                cpb[1].wait()
                combine(gp_b, gq_b)
                scb = scatter(2 * i + 1, gp_b, sem_sb)
                sca.wait()
                scb.wait()

        plsc.subcore_barrier()

        pltpu.sync_copy(acc_sh.at[pl.ds(row0, RPS), :],
                        agg_hbm.at[cid].at[pl.ds(row0, RPS), :])

    return k(p, q, dst3, src3, zeros)


def _head_call(h_parts, deg_parts, wf1, bf1_2d, wf2_row, bf2_2d):
    bn = 1024

    def body(h_ref, deg_ref, w1_ref, b1_ref, w2_ref, b2_ref, o_ref):
        xb = jnp.concatenate([h_ref[0], h_ref[1]], axis=1)
        d = deg_ref[0][:, :1] + deg_ref[1][:, :1]
        xb = xb * (1.0 / jnp.maximum(d, 1.0))
        a = jnp.maximum(
            jnp.dot(xb, w1_ref[...], preferred_element_type=jnp.float32)
            + b1_ref[...],
            0.0,
        )
        z = jnp.sum(a * w2_ref[...], axis=1, keepdims=True) + b2_ref[0, 0]
        o_ref[...] = 1.0 / (1.0 + jnp.exp(-z))

    return pl.pallas_call(
        body,
        grid=(NPAD // bn,),
        in_specs=[
            pl.BlockSpec((NC, bn, H3 // 2), lambda i: (0, i, 0)),
            pl.BlockSpec((NC, bn, DEG_W), lambda i: (0, i, 0)),
            pl.BlockSpec((H3, H_FC), lambda i: (0, 0)),
            pl.BlockSpec((1, H_FC), lambda i: (0, 0)),
            pl.BlockSpec((1, H_FC), lambda i: (0, 0)),
            pl.BlockSpec((1, 1), lambda i: (0, 0)),
        ],
        out_specs=pl.BlockSpec((bn, 1), lambda i: (i, 0)),
        out_shape=jax.ShapeDtypeStruct((NPAD, 1), jnp.float32),
    )(h_parts, deg_parts, wf1, bf1_2d, wf2_row, bf2_2d)


def _split(t):
    hh = t.shape[1] // 2
    return jnp.transpose(t.reshape(NPAD, NC, hh), (1, 0, 2))


def _bcat(b, h):
    return jnp.concatenate([b, jnp.zeros_like(b)]).reshape(1, 2 * h)


def kernel(X, edge_index, W1, b1, W2, b2, W3, b3, Wf1, bf1, Wf2, bf2):
    ei = edge_index.astype(jnp.int32)
    d32 = ei[1].reshape(NC * NS, 25, 5, 80)
    s32 = ei[0].reshape(NC * NS, 25, 5, 80)
    d16 = ei[1].reshape(NS, 25, 10, 80)
    s16 = ei[0].reshape(NS, 25, 10, 80)
    z128 = jnp.zeros((NPAD, ACC_W), jnp.float32)
    xpad = jnp.pad(X, ((0, NPAD - N), (0, 0)))

    t = _table_call(xpad, W1, _bcat(b1, H1), None, 128, H1)
    agg = _edge_combined_call(t, d32, s32, z128, H1, 80, 5, 25,
                              feature_split=False, with_ones=True)
    deg = agg[:, :, H1:H1 + DEG_W]

    t = _table_call(agg[:, :, :H1], W2, _bcat(b2, H2), deg, H1, H2)
    t2 = jnp.stack([
        jnp.concatenate([t[:, :H1], t[:, H2:H2 + H1]], axis=1),
        jnp.concatenate([t[:, H1:H2], t[:, H2 + H1:]], axis=1),
    ])
    agg = _edge_combined_call(t2, d16, s16, z128, H1, 80, 10, 25,
                              feature_split=True, with_ones=False)
    x3 = jnp.concatenate([agg[0, :, :H1], agg[1, :, :H1]], axis=1)

    t = _table_call(x3, W3, _bcat(b3, H3), deg, H2, H3)
    agg = _edge_split_call(_split(t[:, :H3]), _split(t[:, H3:]), d16, s16,
                           z128)

    out = _head_call(
        agg, deg, Wf1, bf1.reshape(1, -1),
        Wf2.reshape(1, -1), bf2.reshape(1, 1),
    )
    return out[:N, 0]


# speedup vs baseline: 5.6638x; 1.0489x over previous
import functools

import jax
import jax.numpy as jnp
from jax import lax
from jax.experimental import pallas as pl
from jax.experimental.pallas import tpu as pltpu
from jax.experimental.pallas import tpu_sc as plsc

N = 10000
NPAD = 10240
E = 320000
NC = 2
NS = 16
LANES = 16
RPS = NPAD // NS
DEG_W = 16
ACC_W = 128

H1, H2, H3, H_FC = 64, 128, 256, 256


def _table_call(x_parts, w, b2d, deg_parts, h_in, h_out):
    bn = 1024
    use_deg = deg_parts is not None
    parts = x_parts.ndim == 3

    def body(*refs):
        if use_deg:
            x_ref, w_ref, b_ref, deg_ref, t_ref = refs
        else:
            x_ref, w_ref, b_ref, t_ref = refs
        xb = (x_ref[0] + x_ref[1]) if parts else x_ref[...]
        if use_deg:
            d = deg_ref[0][:, :1] + deg_ref[1][:, :1]
            xb = xb * (1.0 / jnp.maximum(d, 1.0))
        wa = w_ref[:h_in, :]
        wb = w_ref[h_in:, :]
        wcat = jnp.concatenate([wa - wb, wb], axis=1)
        t_ref[...] = (
            jnp.dot(xb, wcat, preferred_element_type=jnp.float32)
            + b_ref[...]
        )

    xspec = (
        pl.BlockSpec((NC, bn, h_in), lambda i: (0, i, 0))
        if parts
        else pl.BlockSpec((bn, h_in), lambda i: (i, 0))
    )
    in_specs = [
        xspec,
        pl.BlockSpec((2 * h_in, h_out), lambda i: (0, 0)),
        pl.BlockSpec((1, 2 * h_out), lambda i: (0, 0)),
    ]
    args = [x_parts, w, b2d]
    if use_deg:
        in_specs.append(pl.BlockSpec((NC, bn, DEG_W), lambda i: (0, i, 0)))
        args.append(deg_parts)
    return pl.pallas_call(
        body,
        grid=(NPAD // bn,),
        in_specs=in_specs,
        out_specs=pl.BlockSpec((bn, 2 * h_out), lambda i: (i, 0)),
        out_shape=jax.ShapeDtypeStruct((NPAD, 2 * h_out), jnp.float32),
    )(*args)


def _edge_combined_call(t, dst4, src4, zeros, h, chunk, iblk, nblk,
                        feature_split, with_ones):
    mesh = plsc.VectorSubcoreMesh(core_axis_name="c", subcore_axis_name="s")

    scratch = [
        pltpu.VMEM((iblk, chunk), jnp.int32),
        pltpu.VMEM((iblk, chunk), jnp.int32),
        pltpu.VMEM((chunk, 2 * h), jnp.float32),
        pltpu.VMEM((chunk, 2 * h), jnp.float32),
        pltpu.VMEM((chunk, 2 * h), jnp.float32),
        pltpu.VMEM((chunk, 2 * h), jnp.float32),
        pltpu.VMEM_SHARED((NPAD, ACC_W), jnp.float32),
        pltpu.SemaphoreType.DMA,
        pltpu.SemaphoreType.DMA,
        pltpu.SemaphoreType.DMA,
        pltpu.SemaphoreType.DMA,
        pltpu.SemaphoreType.DMA,
        pltpu.SemaphoreType.DMA,
    ]

    @functools.partial(
        pl.kernel,
        out_type=jax.ShapeDtypeStruct((NC, NPAD, ACC_W), jnp.float32),
        mesh=mesh,
        scratch_types=scratch,
    )
    def k(t_hbm, dst_hbm, src_hbm, z_hbm, agg_hbm,
          dst_v, src_v, gd_a, gs_a, gd_b, gs_b,
          acc_sh, sem_ad, sem_as, sem_bd, sem_bs, sem_sa, sem_sb):
        cid = lax.axis_index("c")
        sid = lax.axis_index("s")
        wid = sid if feature_split else cid * NS + sid
        tbl = t_hbm.at[cid] if feature_split else t_hbm
        row0 = sid * RPS

        pltpu.sync_copy(z_hbm.at[pl.ds(row0, RPS), :],
                        acc_sh.at[pl.ds(row0, RPS), :])
        plsc.subcore_barrier()

        def issue(ch, gd, gs, semd, sems):
            return (pltpu.async_copy(tbl.at[dst_v.at[ch]], gd, semd),
                    pltpu.async_copy(tbl.at[src_v.at[ch]], gs, sems))

        def combine(gd, gs):
            @pl.loop(0, chunk)
            def _(r):
                for c in range(h // LANES):
                    sl = pl.ds(c * LANES, LANES)
                    gd[r, sl] = jnp.maximum(
                        gd[r, sl] + gs[r, pl.ds(h + c * LANES, LANES)],
                        0.0,
                    )
                if with_ones:
                    gd[r, pl.ds(h, LANES)] = jnp.full(
                        (LANES,), 1.0, jnp.float32)

        def scatter(ch, gd, sem):
            return pltpu.async_copy(gd, acc_sh.at[dst_v.at[ch]], sem,
                                    add=True)

        @pl.loop(0, nblk)
        def _(blk):
            pltpu.sync_copy(dst_hbm.at[wid].at[blk], dst_v)
            pltpu.sync_copy(src_hbm.at[wid].at[blk], src_v)

            @pl.loop(0, iblk // 2)
            def _(i):
                cpa = issue(2 * i, gd_a, gs_a, sem_ad, sem_as)
                cpb = issue(2 * i + 1, gd_b, gs_b, sem_bd, sem_bs)
                cpa[0].wait()
                cpa[1].wait()
                combine(gd_a, gs_a)
                sca = scatter(2 * i, gd_a, sem_sa)
                cpb[0].wait()
                cpb[1].wait()
                combine(gd_b, gs_b)
                scb = scatter(2 * i + 1, gd_b, sem_sb)
                sca.wait()
                scb.wait()

            if iblk % 2:
                cpa = issue(iblk - 1, gd_a, gs_a, sem_ad, sem_as)
                cpa[0].wait()
                cpa[1].wait()
                combine(gd_a, gs_a)
                scatter(iblk - 1, gd_a, sem_sa).wait()

        plsc.subcore_barrier()

        pltpu.sync_copy(acc_sh.at[pl.ds(row0, RPS), :],
                        agg_hbm.at[cid].at[pl.ds(row0, RPS), :])

    return k(t, dst4, src4, zeros)


def _edge_split_call(p, q, dst3, src3, zeros):
    hh = p.shape[2]
    nblk, iblk, chunk = dst3.shape[1], dst3.shape[2], dst3.shape[3]
    mesh = plsc.VectorSubcoreMesh(core_axis_name="c", subcore_axis_name="s")

    @functools.partial(
        pl.kernel,
        out_type=jax.ShapeDtypeStruct((NC, NPAD, hh), jnp.float32),
        mesh=mesh,
        scratch_types=[
            pltpu.VMEM((iblk, chunk), jnp.int32),
            pltpu.VMEM((iblk, chunk), jnp.int32),
            pltpu.VMEM((chunk, hh), jnp.float32),
            pltpu.VMEM((chunk, hh), jnp.float32),
            pltpu.VMEM((chunk, hh), jnp.float32),
            pltpu.VMEM((chunk, hh), jnp.float32),
            pltpu.VMEM_SHARED((NPAD, hh), jnp.float32),
            pltpu.SemaphoreType.DMA,
            pltpu.SemaphoreType.DMA,
            pltpu.SemaphoreType.DMA,
            pltpu.SemaphoreType.DMA,
            pltpu.SemaphoreType.DMA,
            pltpu.SemaphoreType.DMA,
        ],
    )
    def k(p_hbm, q_hbm, dst_hbm, src_hbm, z_hbm, agg_hbm,
          dst_v, src_v, gp_a, gq_a, gp_b, gq_b, acc_sh,
          sem_ad, sem_as, sem_bd, sem_bs, sem_sa, sem_sb):
        cid = lax.axis_index("c")
        sid = lax.axis_index("s")
        row0 = sid * RPS

        pltpu.sync_copy(z_hbm.at[pl.ds(row0, RPS), :],
                        acc_sh.at[pl.ds(row0, RPS), :])
        plsc.subcore_barrier()

        def issue(ch, gp, gq, semd, sems):
            return (pltpu.async_copy(p_hbm.at[cid].at[dst_v.at[ch]], gp,
                                     semd),
                    pltpu.async_copy(q_hbm.at[cid].at[src_v.at[ch]], gq,
                                     sems))

        def combine(gp, gq):
            @pl.loop(0, chunk)
            def _(r):
                for c in range(hh // LANES):
                    sl = pl.ds(c * LANES, LANES)
                    gp[r, sl] = jnp.maximum(gp[r, sl] + gq[r, sl], 0.0)

        def scatter(ch, gp, sem):
            return pltpu.async_copy(gp, acc_sh.at[dst_v.at[ch]], sem,
                                    add=True)

        @pl.loop(0, nblk)
        def _(blk):
            pltpu.sync_copy(dst_hbm.at[sid].at[blk], dst_v)
            pltpu.sync_copy(src_hbm.at[sid].at[blk], src_v)

            @pl.loop(0, iblk // 2)
            def _(i):
                cpa = issue(2 * i, gp_a, gq_a, sem_ad, sem_as)
                cpb = issue(2 * i + 1, gp_b, gq_b, sem_bd, sem_bs)
                cpa[0].wait()
                cpa[1].wait()
                combine(gp_a, gq_a)
                sca = scatter(2 * i, gp_a, sem_sa)
                cpb[0].wait()
                cpb[1].wait()
                combine(gp_b, gq_b)
                scb = scatter(2 * i + 1, gp_b, sem_sb)
                sca.wait()
                scb.wait()

            if iblk % 2:
                cpa = issue(iblk - 1, gp_a, gq_a, sem_ad, sem_as)
                cpa[0].wait()
                cpa[1].wait()
                combine(gp_a, gq_a)
                scatter(iblk - 1, gp_a, sem_sa).wait()

        plsc.subcore_barrier()

        pltpu.sync_copy(acc_sh.at[pl.ds(row0, RPS), :],
                        agg_hbm.at[cid].at[pl.ds(row0, RPS), :])

    return k(p, q, dst3, src3, zeros)


def _head_call(h_parts, deg_parts, wf1, bf1_2d, wf2_row, bf2_2d):
    bn = 1024

    def body(h_ref, deg_ref, w1_ref, b1_ref, w2_ref, b2_ref, o_ref):
        xb = jnp.concatenate([h_ref[0], h_ref[1]], axis=1)
        d = deg_ref[0][:, :1] + deg_ref[1][:, :1]
        xb = xb * (1.0 / jnp.maximum(d, 1.0))
        a = jnp.maximum(
            jnp.dot(xb, w1_ref[...], preferred_element_type=jnp.float32)
            + b1_ref[...],
            0.0,
        )
        z = jnp.sum(a * w2_ref[...], axis=1, keepdims=True) + b2_ref[0, 0]
        o_ref[...] = 1.0 / (1.0 + jnp.exp(-z))

    return pl.pallas_call(
        body,
        grid=(NPAD // bn,),
        in_specs=[
            pl.BlockSpec((NC, bn, H3 // 2), lambda i: (0, i, 0)),
            pl.BlockSpec((NC, bn, DEG_W), lambda i: (0, i, 0)),
            pl.BlockSpec((H3, H_FC), lambda i: (0, 0)),
            pl.BlockSpec((1, H_FC), lambda i: (0, 0)),
            pl.BlockSpec((1, H_FC), lambda i: (0, 0)),
            pl.BlockSpec((1, 1), lambda i: (0, 0)),
        ],
        out_specs=pl.BlockSpec((bn, 1), lambda i: (i, 0)),
        out_shape=jax.ShapeDtypeStruct((NPAD, 1), jnp.float32),
    )(h_parts, deg_parts, wf1, bf1_2d, wf2_row, bf2_2d)


def _split(t):
    hh = t.shape[1] // 2
    return jnp.transpose(t.reshape(NPAD, NC, hh), (1, 0, 2))


def _bcat(b, h):
    return jnp.concatenate([b, jnp.zeros_like(b)]).reshape(1, 2 * h)


def kernel(X, edge_index, W1, b1, W2, b2, W3, b3, Wf1, bf1, Wf2, bf2):
    ei = edge_index.astype(jnp.int32)
    d32 = ei[1].reshape(NC * NS, 5, 25, 80)
    s32 = ei[0].reshape(NC * NS, 5, 25, 80)
    d16 = ei[1].reshape(NS, 10, 25, 80)
    s16 = ei[0].reshape(NS, 10, 25, 80)
    z128 = jnp.zeros((NPAD, ACC_W), jnp.float32)
    xpad = jnp.pad(X, ((0, NPAD - N), (0, 0)))

    t = _table_call(xpad, W1, _bcat(b1, H1), None, 128, H1)
    agg = _edge_combined_call(t, d32, s32, z128, H1, 80, 25, 5,
                              feature_split=False, with_ones=True)
    deg = agg[:, :, H1:H1 + DEG_W]

    t = _table_call(agg[:, :, :H1], W2, _bcat(b2, H2), deg, H1, H2)
    t2 = jnp.stack([
        jnp.concatenate([t[:, :H1], t[:, H2:H2 + H1]], axis=1),
        jnp.concatenate([t[:, H1:H2], t[:, H2 + H1:]], axis=1),
    ])
    agg = _edge_combined_call(t2, d16, s16, z128, H1, 80, 25, 10,
                              feature_split=True, with_ones=False)
    x3 = jnp.concatenate([agg[0, :, :H1], agg[1, :, :H1]], axis=1)

    t = _table_call(x3, W3, _bcat(b3, H3), deg, H2, H3)
    agg = _edge_split_call(_split(t[:, :H3]), _split(t[:, H3:]), d16, s16,
                           z128)

    out = _head_call(
        agg, deg, Wf1, bf1.reshape(1, -1),
        Wf2.reshape(1, -1), bf2.reshape(1, 1),
    )
    return out[:N, 0]


# speedup vs baseline: 5.9639x; 1.0530x over previous
import functools

import jax
import jax.numpy as jnp
from jax import lax
from jax.experimental import pallas as pl
from jax.experimental.pallas import tpu as pltpu
from jax.experimental.pallas import tpu_sc as plsc

N = 10000
NPAD = 10240
E = 320000
NC = 2
NS = 16
LANES = 16
RPS = NPAD // NS
DEG_W = 16
ACC_W = 128

H1, H2, H3, H_FC = 64, 128, 256, 256


def _table_call(x_parts, w, b2d, deg_parts, h_in, h_out, out_mode="flat"):
    bn = 1024
    use_deg = deg_parts is not None
    parts = x_parts.ndim == 3

    def body(*refs):
        if use_deg:
            x_ref, w_ref, b_ref, deg_ref, *outs = refs
        else:
            x_ref, w_ref, b_ref, *outs = refs
        xb = (x_ref[0] + x_ref[1]) if parts else x_ref[...]
        if use_deg:
            d = deg_ref[0][:, :1] + deg_ref[1][:, :1]
            xb = xb * (1.0 / jnp.maximum(d, 1.0))
        wa = w_ref[:h_in, :]
        wb = w_ref[h_in:, :]
        wcat = jnp.concatenate([wa - wb, wb], axis=1)
        full = (
            jnp.dot(xb, wcat, preferred_element_type=jnp.float32)
            + b_ref[...]
        )
        if out_mode == "flat":
            outs[0][...] = full
        elif out_mode == "pc64":
            hh = h_out // 2
            outs[0][0] = jnp.concatenate(
                [full[:, :hh], full[:, h_out:h_out + hh]], axis=1)
            outs[0][1] = jnp.concatenate(
                [full[:, hh:h_out], full[:, h_out + hh:]], axis=1)
        else:
            hh = h_out // 2
            outs[0][0] = full[:, :hh]
            outs[0][1] = full[:, hh:h_out]
            outs[1][0] = full[:, h_out:h_out + hh]
            outs[1][1] = full[:, h_out + hh:]

    xspec = (
        pl.BlockSpec((NC, bn, h_in), lambda i: (0, i, 0))
        if parts
        else pl.BlockSpec((bn, h_in), lambda i: (i, 0))
    )
    in_specs = [
        xspec,
        pl.BlockSpec((2 * h_in, h_out), lambda i: (0, 0)),
        pl.BlockSpec((1, 2 * h_out), lambda i: (0, 0)),
    ]
    args = [x_parts, w, b2d]
    if use_deg:
        in_specs.append(pl.BlockSpec((NC, bn, DEG_W), lambda i: (0, i, 0)))
        args.append(deg_parts)
    if out_mode == "flat":
        out_specs = pl.BlockSpec((bn, 2 * h_out), lambda i: (i, 0))
        out_shape = jax.ShapeDtypeStruct((NPAD, 2 * h_out), jnp.float32)
    elif out_mode == "pc64":
        out_specs = pl.BlockSpec((NC, bn, h_out), lambda i: (0, i, 0))
        out_shape = jax.ShapeDtypeStruct((NC, NPAD, h_out), jnp.float32)
    else:
        out_specs = [
            pl.BlockSpec((NC, bn, h_out // 2), lambda i: (0, i, 0)),
            pl.BlockSpec((NC, bn, h_out // 2), lambda i: (0, i, 0)),
        ]
        out_shape = [
            jax.ShapeDtypeStruct((NC, NPAD, h_out // 2), jnp.float32),
            jax.ShapeDtypeStruct((NC, NPAD, h_out // 2), jnp.float32),
        ]
    return pl.pallas_call(
        body,
        grid=(NPAD // bn,),
        in_specs=in_specs,
        out_specs=out_specs,
        out_shape=out_shape,
    )(*args)


def _edge_combined_call(t, dst4, src4, zeros, h, chunk, iblk, nblk,
                        feature_split, with_ones):
    mesh = plsc.VectorSubcoreMesh(core_axis_name="c", subcore_axis_name="s")

    scratch = [
        pltpu.VMEM((iblk, chunk), jnp.int32),
        pltpu.VMEM((iblk, chunk), jnp.int32),
        pltpu.VMEM((chunk, 2 * h), jnp.float32),
        pltpu.VMEM((chunk, 2 * h), jnp.float32),
        pltpu.VMEM((chunk, 2 * h), jnp.float32),
        pltpu.VMEM((chunk, 2 * h), jnp.float32),
        pltpu.VMEM_SHARED((NPAD, ACC_W), jnp.float32),
        pltpu.SemaphoreType.DMA,
        pltpu.SemaphoreType.DMA,
        pltpu.SemaphoreType.DMA,
        pltpu.SemaphoreType.DMA,
        pltpu.SemaphoreType.DMA,
        pltpu.SemaphoreType.DMA,
    ]

    @functools.partial(
        pl.kernel,
        out_type=jax.ShapeDtypeStruct((NC, NPAD, ACC_W), jnp.float32),
        mesh=mesh,
        scratch_types=scratch,
    )
    def k(t_hbm, dst_hbm, src_hbm, z_hbm, agg_hbm,
          dst_v, src_v, gd_a, gs_a, gd_b, gs_b,
          acc_sh, sem_ad, sem_as, sem_bd, sem_bs, sem_sa, sem_sb):
        cid = lax.axis_index("c")
        sid = lax.axis_index("s")
        wid = sid if feature_split else cid * NS + sid
        tbl = t_hbm.at[cid] if feature_split else t_hbm
        row0 = sid * RPS

        pltpu.sync_copy(z_hbm.at[pl.ds(row0, RPS), :],
                        acc_sh.at[pl.ds(row0, RPS), :])
        plsc.subcore_barrier()

        def issue(ch, gd, gs, semd, sems):
            return (pltpu.async_copy(tbl.at[dst_v.at[ch]], gd, semd),
                    pltpu.async_copy(tbl.at[src_v.at[ch]], gs, sems))

        def combine(gd, gs):
            @pl.loop(0, chunk)
            def _(r):
                for c in range(h // LANES):
                    sl = pl.ds(c * LANES, LANES)
                    gd[r, sl] = jnp.maximum(
                        gd[r, sl] + gs[r, pl.ds(h + c * LANES, LANES)],
                        0.0,
                    )
                if with_ones:
                    gd[r, pl.ds(h, LANES)] = jnp.full(
                        (LANES,), 1.0, jnp.float32)

        def scatter(ch, gd, sem):
            return pltpu.async_copy(gd, acc_sh.at[dst_v.at[ch]], sem,
                                    add=True)

        @pl.loop(0, nblk)
        def _(blk):
            pltpu.sync_copy(dst_hbm.at[wid].at[blk], dst_v)
            pltpu.sync_copy(src_hbm.at[wid].at[blk], src_v)

            @pl.loop(0, iblk // 2)
            def _(i):
                cpa = issue(2 * i, gd_a, gs_a, sem_ad, sem_as)
                cpb = issue(2 * i + 1, gd_b, gs_b, sem_bd, sem_bs)
                cpa[0].wait()
                cpa[1].wait()
                combine(gd_a, gs_a)
                sca = scatter(2 * i, gd_a, sem_sa)
                cpb[0].wait()
                cpb[1].wait()
                combine(gd_b, gs_b)
                scb = scatter(2 * i + 1, gd_b, sem_sb)
                sca.wait()
                scb.wait()

            if iblk % 2:
                cpa = issue(iblk - 1, gd_a, gs_a, sem_ad, sem_as)
                cpa[0].wait()
                cpa[1].wait()
                combine(gd_a, gs_a)
                scatter(iblk - 1, gd_a, sem_sa).wait()

        plsc.subcore_barrier()

        pltpu.sync_copy(acc_sh.at[pl.ds(row0, RPS), :],
                        agg_hbm.at[cid].at[pl.ds(row0, RPS), :])

    return k(t, dst4, src4, zeros)


def _edge_split_call(p, q, dst3, src3, zeros):
    hh = p.shape[2]
    nblk, iblk, chunk = dst3.shape[1], dst3.shape[2], dst3.shape[3]
    mesh = plsc.VectorSubcoreMesh(core_axis_name="c", subcore_axis_name="s")

    @functools.partial(
        pl.kernel,
        out_type=jax.ShapeDtypeStruct((NC, NPAD, hh), jnp.float32),
        mesh=mesh,
        scratch_types=[
            pltpu.VMEM((iblk, chunk), jnp.int32),
            pltpu.VMEM((iblk, chunk), jnp.int32),
            pltpu.VMEM((chunk, hh), jnp.float32),
            pltpu.VMEM((chunk, hh), jnp.float32),
            pltpu.VMEM((chunk, hh), jnp.float32),
            pltpu.VMEM((chunk, hh), jnp.float32),
            pltpu.VMEM_SHARED((NPAD, hh), jnp.float32),
            pltpu.SemaphoreType.DMA,
            pltpu.SemaphoreType.DMA,
            pltpu.SemaphoreType.DMA,
            pltpu.SemaphoreType.DMA,
            pltpu.SemaphoreType.DMA,
            pltpu.SemaphoreType.DMA,
        ],
    )
    def k(p_hbm, q_hbm, dst_hbm, src_hbm, z_hbm, agg_hbm,
          dst_v, src_v, gp_a, gq_a, gp_b, gq_b, acc_sh,
          sem_ad, sem_as, sem_bd, sem_bs, sem_sa, sem_sb):
        cid = lax.axis_index("c")
        sid = lax.axis_index("s")
        row0 = sid * RPS

        pltpu.sync_copy(z_hbm.at[pl.ds(row0, RPS), :],
                        acc_sh.at[pl.ds(row0, RPS), :])
        plsc.subcore_barrier()

        def issue(ch, gp, gq, semd, sems):
            return (pltpu.async_copy(p_hbm.at[cid].at[dst_v.at[ch]], gp,
                                     semd),
                    pltpu.async_copy(q_hbm.at[cid].at[src_v.at[ch]], gq,
                                     sems))

        def combine(gp, gq):
            @pl.loop(0, chunk)
            def _(r):
                for c in range(hh // LANES):
                    sl = pl.ds(c * LANES, LANES)
                    gp[r, sl] = jnp.maximum(gp[r, sl] + gq[r, sl], 0.0)

        def scatter(ch, gp, sem):
            return pltpu.async_copy(gp, acc_sh.at[dst_v.at[ch]], sem,
                                    add=True)

        @pl.loop(0, nblk)
        def _(blk):
            pltpu.sync_copy(dst_hbm.at[sid].at[blk], dst_v)
            pltpu.sync_copy(src_hbm.at[sid].at[blk], src_v)

            @pl.loop(0, iblk // 2)
            def _(i):
                cpa = issue(2 * i, gp_a, gq_a, sem_ad, sem_as)
                cpb = issue(2 * i + 1, gp_b, gq_b, sem_bd, sem_bs)
                cpa[0].wait()
                cpa[1].wait()
                combine(gp_a, gq_a)
                sca = scatter(2 * i, gp_a, sem_sa)
                cpb[0].wait()
                cpb[1].wait()
                combine(gp_b, gq_b)
                scb = scatter(2 * i + 1, gp_b, sem_sb)
                sca.wait()
                scb.wait()

            if iblk % 2:
                cpa = issue(iblk - 1, gp_a, gq_a, sem_ad, sem_as)
                cpa[0].wait()
                cpa[1].wait()
                combine(gp_a, gq_a)
                scatter(iblk - 1, gp_a, sem_sa).wait()

        plsc.subcore_barrier()

        pltpu.sync_copy(acc_sh.at[pl.ds(row0, RPS), :],
                        agg_hbm.at[cid].at[pl.ds(row0, RPS), :])

    return k(p, q, dst3, src3, zeros)


def _head_call(h_parts, deg_parts, wf1, bf1_2d, wf2_row, bf2_2d):
    bn = 1024

    def body(h_ref, deg_ref, w1_ref, b1_ref, w2_ref, b2_ref, o_ref):
        xb = jnp.concatenate([h_ref[0], h_ref[1]], axis=1)
        d = deg_ref[0][:, :1] + deg_ref[1][:, :1]
        xb = xb * (1.0 / jnp.maximum(d, 1.0))
        a = jnp.maximum(
            jnp.dot(xb, w1_ref[...], preferred_element_type=jnp.float32)
            + b1_ref[...],
            0.0,
        )
        z = jnp.sum(a * w2_ref[...], axis=1, keepdims=True) + b2_ref[0, 0]
        o_ref[...] = 1.0 / (1.0 + jnp.exp(-z))

    return pl.pallas_call(
        body,
        grid=(NPAD // bn,),
        in_specs=[
            pl.BlockSpec((NC, bn, H3 // 2), lambda i: (0, i, 0)),
            pl.BlockSpec((NC, bn, DEG_W), lambda i: (0, i, 0)),
            pl.BlockSpec((H3, H_FC), lambda i: (0, 0)),
            pl.BlockSpec((1, H_FC), lambda i: (0, 0)),
            pl.BlockSpec((1, H_FC), lambda i: (0, 0)),
            pl.BlockSpec((1, 1), lambda i: (0, 0)),
        ],
        out_specs=pl.BlockSpec((bn, 1), lambda i: (i, 0)),
        out_shape=jax.ShapeDtypeStruct((NPAD, 1), jnp.float32),
    )(h_parts, deg_parts, wf1, bf1_2d, wf2_row, bf2_2d)


def _bcat(b, h):
    return jnp.concatenate([b, jnp.zeros_like(b)]).reshape(1, 2 * h)


def kernel(X, edge_index, W1, b1, W2, b2, W3, b3, Wf1, bf1, Wf2, bf2):
    ei = edge_index.astype(jnp.int32)
    d32 = ei[1].reshape(NC * NS, 5, 25, 80)
    s32 = ei[0].reshape(NC * NS, 5, 25, 80)
    d16 = ei[1].reshape(NS, 10, 25, 80)
    s16 = ei[0].reshape(NS, 10, 25, 80)
    z128 = jnp.zeros((NPAD, ACC_W), jnp.float32)
    xpad = jnp.pad(X, ((0, NPAD - N), (0, 0)))

    t = _table_call(xpad, W1, _bcat(b1, H1), None, 128, H1)
    agg = _edge_combined_call(t, d32, s32, z128, H1, 80, 25, 5,
                              feature_split=False, with_ones=True)
    deg = agg[:, :, H1:H1 + DEG_W]

    t2 = _table_call(agg[:, :, :H1], W2, _bcat(b2, H2), deg, H1, H2,
                     out_mode="pc64")
    agg = _edge_combined_call(t2, d16, s16, z128, H1, 80, 25, 10,
                              feature_split=True, with_ones=False)
    x3 = jnp.concatenate([agg[0, :, :H1], agg[1, :, :H1]], axis=1)

    p3, q3 = _table_call(x3, W3, _bcat(b3, H3), deg, H2, H3,
                         out_mode="pc128")
    agg = _edge_split_call(p3, q3, d16, s16, z128)

    out = _head_call(
        agg, deg, Wf1, bf1.reshape(1, -1),
        Wf2.reshape(1, -1), bf2.reshape(1, 1),
    )
    return out[:N, 0]


# speedup vs baseline: 5.9933x; 1.0049x over previous
import functools

import jax
import jax.numpy as jnp
from jax import lax
from jax.experimental import pallas as pl
from jax.experimental.pallas import tpu as pltpu
from jax.experimental.pallas import tpu_sc as plsc

N = 10000
NPAD = 10240
E = 320000
NC = 2
NS = 16
LANES = 16
RPS = NPAD // NS
DEG_W = 16
ACC_W = 128

H1, H2, H3, H_FC = 64, 128, 256, 256


def _table_call(x_parts, w, b2d, deg_parts, h_in, h_out, out_mode="flat",
                x_mode="sum"):
    bn = 1024
    use_deg = deg_parts is not None
    parts = x_parts.ndim == 3
    xw = x_parts.shape[-1]

    def body(*refs):
        if use_deg:
            x_ref, w_ref, b_ref, deg_ref, *outs = refs
        else:
            x_ref, w_ref, b_ref, *outs = refs
        if parts and x_mode == "sum":
            xb = (x_ref[0] + x_ref[1])[:, :h_in]
        elif parts:
            xb = jnp.concatenate(
                [x_ref[0][:, :h_in // 2], x_ref[1][:, :h_in // 2]], axis=1)
        else:
            xb = x_ref[...]
        if use_deg:
            d = deg_ref[0][:, :1] + deg_ref[1][:, :1]
            xb = xb * (1.0 / jnp.maximum(d, 1.0))
        wa = w_ref[:h_in, :]
        wb = w_ref[h_in:, :]
        wcat = jnp.concatenate([wa - wb, wb], axis=1)
        full = (
            jnp.dot(xb, wcat, preferred_element_type=jnp.float32)
            + b_ref[...]
        )
        if out_mode == "flat":
            outs[0][...] = full
        elif out_mode == "pc64":
            hh = h_out // 2
            outs[0][0] = jnp.concatenate(
                [full[:, :hh], full[:, h_out:h_out + hh]], axis=1)
            outs[0][1] = jnp.concatenate(
                [full[:, hh:h_out], full[:, h_out + hh:]], axis=1)
        else:
            hh = h_out // 2
            outs[0][0] = full[:, :hh]
            outs[0][1] = full[:, hh:h_out]
            outs[1][0] = full[:, h_out:h_out + hh]
            outs[1][1] = full[:, h_out + hh:]

    xspec = (
        pl.BlockSpec((NC, bn, xw), lambda i: (0, i, 0))
        if parts
        else pl.BlockSpec((bn, xw), lambda i: (i, 0))
    )
    in_specs = [
        xspec,
        pl.BlockSpec((2 * h_in, h_out), lambda i: (0, 0)),
        pl.BlockSpec((1, 2 * h_out), lambda i: (0, 0)),
    ]
    args = [x_parts, w, b2d]
    if use_deg:
        in_specs.append(pl.BlockSpec((NC, bn, DEG_W), lambda i: (0, i, 0)))
        args.append(deg_parts)
    if out_mode == "flat":
        out_specs = pl.BlockSpec((bn, 2 * h_out), lambda i: (i, 0))
        out_shape = jax.ShapeDtypeStruct((NPAD, 2 * h_out), jnp.float32)
    elif out_mode == "pc64":
        out_specs = pl.BlockSpec((NC, bn, h_out), lambda i: (0, i, 0))
        out_shape = jax.ShapeDtypeStruct((NC, NPAD, h_out), jnp.float32)
    else:
        out_specs = [
            pl.BlockSpec((NC, bn, h_out // 2), lambda i: (0, i, 0)),
            pl.BlockSpec((NC, bn, h_out // 2), lambda i: (0, i, 0)),
        ]
        out_shape = [
            jax.ShapeDtypeStruct((NC, NPAD, h_out // 2), jnp.float32),
            jax.ShapeDtypeStruct((NC, NPAD, h_out // 2), jnp.float32),
        ]
    return pl.pallas_call(
        body,
        grid=(NPAD // bn,),
        in_specs=in_specs,
        out_specs=out_specs,
        out_shape=out_shape,
    )(*args)


def _edge_combined_call(t, dst4, src4, zeros, h, chunk, iblk, nblk,
                        feature_split, with_ones):
    mesh = plsc.VectorSubcoreMesh(core_axis_name="c", subcore_axis_name="s")

    scratch = [
        pltpu.VMEM((iblk, chunk), jnp.int32),
        pltpu.VMEM((iblk, chunk), jnp.int32),
        pltpu.VMEM((chunk, 2 * h), jnp.float32),
        pltpu.VMEM((chunk, 2 * h), jnp.float32),
        pltpu.VMEM((chunk, 2 * h), jnp.float32),
        pltpu.VMEM((chunk, 2 * h), jnp.float32),
        pltpu.VMEM_SHARED((NPAD, ACC_W), jnp.float32),
        pltpu.SemaphoreType.DMA,
        pltpu.SemaphoreType.DMA,
        pltpu.SemaphoreType.DMA,
        pltpu.SemaphoreType.DMA,
        pltpu.SemaphoreType.DMA,
        pltpu.SemaphoreType.DMA,
    ]

    @functools.partial(
        pl.kernel,
        out_type=jax.ShapeDtypeStruct((NC, NPAD, ACC_W), jnp.float32),
        mesh=mesh,
        scratch_types=scratch,
    )
    def k(t_hbm, dst_hbm, src_hbm, z_hbm, agg_hbm,
          dst_v, src_v, gd_a, gs_a, gd_b, gs_b,
          acc_sh, sem_ad, sem_as, sem_bd, sem_bs, sem_sa, sem_sb):
        cid = lax.axis_index("c")
        sid = lax.axis_index("s")
        wid = sid if feature_split else cid * NS + sid
        tbl = t_hbm.at[cid] if feature_split else t_hbm
        row0 = sid * RPS

        pltpu.sync_copy(z_hbm.at[pl.ds(row0, RPS), :],
                        acc_sh.at[pl.ds(row0, RPS), :])
        plsc.subcore_barrier()

        def issue(ch, gd, gs, semd, sems):
            return (pltpu.async_copy(tbl.at[dst_v.at[ch]], gd, semd),
                    pltpu.async_copy(tbl.at[src_v.at[ch]], gs, sems))

        def combine(gd, gs):
            @pl.loop(0, chunk)
            def _(r):
                for c in range(h // LANES):
                    sl = pl.ds(c * LANES, LANES)
                    gd[r, sl] = jnp.maximum(
                        gd[r, sl] + gs[r, pl.ds(h + c * LANES, LANES)],
                        0.0,
                    )
                if with_ones:
                    gd[r, pl.ds(h, LANES)] = jnp.full(
                        (LANES,), 1.0, jnp.float32)

        def scatter(ch, gd, sem):
            return pltpu.async_copy(gd, acc_sh.at[dst_v.at[ch]], sem,
                                    add=True)

        @pl.loop(0, nblk)
        def _(blk):
            pltpu.sync_copy(dst_hbm.at[wid].at[blk], dst_v)
            pltpu.sync_copy(src_hbm.at[wid].at[blk], src_v)

            @pl.loop(0, iblk // 2)
            def _(i):
                cpa = issue(2 * i, gd_a, gs_a, sem_ad, sem_as)
                cpb = issue(2 * i + 1, gd_b, gs_b, sem_bd, sem_bs)
                cpa[0].wait()
                cpa[1].wait()
                combine(gd_a, gs_a)
                sca = scatter(2 * i, gd_a, sem_sa)
                cpb[0].wait()
                cpb[1].wait()
                combine(gd_b, gs_b)
                scb = scatter(2 * i + 1, gd_b, sem_sb)
                sca.wait()
                scb.wait()

            if iblk % 2:
                cpa = issue(iblk - 1, gd_a, gs_a, sem_ad, sem_as)
                cpa[0].wait()
                cpa[1].wait()
                combine(gd_a, gs_a)
                scatter(iblk - 1, gd_a, sem_sa).wait()

        plsc.subcore_barrier()

        pltpu.sync_copy(acc_sh.at[pl.ds(row0, RPS), :],
                        agg_hbm.at[cid].at[pl.ds(row0, RPS), :])

    return k(t, dst4, src4, zeros)


def _edge_split_call(p, q, dst3, src3, zeros):
    hh = p.shape[2]
    nblk, iblk, chunk = dst3.shape[1], dst3.shape[2], dst3.shape[3]
    mesh = plsc.VectorSubcoreMesh(core_axis_name="c", subcore_axis_name="s")

    @functools.partial(
        pl.kernel,
        out_type=jax.ShapeDtypeStruct((NC, NPAD, hh), jnp.float32),
        mesh=mesh,
        scratch_types=[
            pltpu.VMEM((iblk, chunk), jnp.int32),
            pltpu.VMEM((iblk, chunk), jnp.int32),
            pltpu.VMEM((chunk, hh), jnp.float32),
            pltpu.VMEM((chunk, hh), jnp.float32),
            pltpu.VMEM((chunk, hh), jnp.float32),
            pltpu.VMEM((chunk, hh), jnp.float32),
            pltpu.VMEM_SHARED((NPAD, hh), jnp.float32),
            pltpu.SemaphoreType.DMA,
            pltpu.SemaphoreType.DMA,
            pltpu.SemaphoreType.DMA,
            pltpu.SemaphoreType.DMA,
            pltpu.SemaphoreType.DMA,
            pltpu.SemaphoreType.DMA,
        ],
    )
    def k(p_hbm, q_hbm, dst_hbm, src_hbm, z_hbm, agg_hbm,
          dst_v, src_v, gp_a, gq_a, gp_b, gq_b, acc_sh,
          sem_ad, sem_as, sem_bd, sem_bs, sem_sa, sem_sb):
        cid = lax.axis_index("c")
        sid = lax.axis_index("s")
        row0 = sid * RPS

        pltpu.sync_copy(z_hbm.at[pl.ds(row0, RPS), :],
                        acc_sh.at[pl.ds(row0, RPS), :])
        plsc.subcore_barrier()

        def issue(ch, gp, gq, semd, sems):
            return (pltpu.async_copy(p_hbm.at[cid].at[dst_v.at[ch]], gp,
                                     semd),
                    pltpu.async_copy(q_hbm.at[cid].at[src_v.at[ch]], gq,
                                     sems))

        def combine(gp, gq):
            @pl.loop(0, chunk)
            def _(r):
                for c in range(hh // LANES):
                    sl = pl.ds(c * LANES, LANES)
                    gp[r, sl] = jnp.maximum(gp[r, sl] + gq[r, sl], 0.0)

        def scatter(ch, gp, sem):
            return pltpu.async_copy(gp, acc_sh.at[dst_v.at[ch]], sem,
                                    add=True)

        @pl.loop(0, nblk)
        def _(blk):
            pltpu.sync_copy(dst_hbm.at[sid].at[blk], dst_v)
            pltpu.sync_copy(src_hbm.at[sid].at[blk], src_v)

            @pl.loop(0, iblk // 2)
            def _(i):
                cpa = issue(2 * i, gp_a, gq_a, sem_ad, sem_as)
                cpb = issue(2 * i + 1, gp_b, gq_b, sem_bd, sem_bs)
                cpa[0].wait()
                cpa[1].wait()
                combine(gp_a, gq_a)
                sca = scatter(2 * i, gp_a, sem_sa)
                cpb[0].wait()
                cpb[1].wait()
                combine(gp_b, gq_b)
                scb = scatter(2 * i + 1, gp_b, sem_sb)
                sca.wait()
                scb.wait()

            if iblk % 2:
                cpa = issue(iblk - 1, gp_a, gq_a, sem_ad, sem_as)
                cpa[0].wait()
                cpa[1].wait()
                combine(gp_a, gq_a)
                scatter(iblk - 1, gp_a, sem_sa).wait()

        plsc.subcore_barrier()

        pltpu.sync_copy(acc_sh.at[pl.ds(row0, RPS), :],
                        agg_hbm.at[cid].at[pl.ds(row0, RPS), :])

    return k(p, q, dst3, src3, zeros)


def _head_call(h_parts, deg_parts, wf1, bf1_2d, wf2_row, bf2_2d):
    bn = 1024

    def body(h_ref, deg_ref, w1_ref, b1_ref, w2_ref, b2_ref, o_ref):
        xb = jnp.concatenate([h_ref[0], h_ref[1]], axis=1)
        d = deg_ref[0][:, :1] + deg_ref[1][:, :1]
        xb = xb * (1.0 / jnp.maximum(d, 1.0))
        a = jnp.maximum(
            jnp.dot(xb, w1_ref[...], preferred_element_type=jnp.float32)
            + b1_ref[...],
            0.0,
        )
        z = jnp.sum(a * w2_ref[...], axis=1, keepdims=True) + b2_ref[0, 0]
        o_ref[...] = 1.0 / (1.0 + jnp.exp(-z))

    return pl.pallas_call(
        body,
        grid=(NPAD // bn,),
        in_specs=[
            pl.BlockSpec((NC, bn, H3 // 2), lambda i: (0, i, 0)),
            pl.BlockSpec((NC, bn, DEG_W), lambda i: (0, i, 0)),
            pl.BlockSpec((H3, H_FC), lambda i: (0, 0)),
            pl.BlockSpec((1, H_FC), lambda i: (0, 0)),
            pl.BlockSpec((1, H_FC), lambda i: (0, 0)),
            pl.BlockSpec((1, 1), lambda i: (0, 0)),
        ],
        out_specs=pl.BlockSpec((bn, 1), lambda i: (i, 0)),
        out_shape=jax.ShapeDtypeStruct((NPAD, 1), jnp.float32),
    )(h_parts, deg_parts, wf1, bf1_2d, wf2_row, bf2_2d)


def _bcat(b, h):
    return jnp.concatenate([b, jnp.zeros_like(b)]).reshape(1, 2 * h)


def kernel(X, edge_index, W1, b1, W2, b2, W3, b3, Wf1, bf1, Wf2, bf2):
    ei = edge_index.astype(jnp.int32)
    d32 = ei[1].reshape(NC * NS, 5, 25, 80)
    s32 = ei[0].reshape(NC * NS, 5, 25, 80)
    d16 = ei[1].reshape(NS, 10, 25, 80)
    s16 = ei[0].reshape(NS, 10, 25, 80)
    z128 = jnp.zeros((NPAD, ACC_W), jnp.float32)
    xpad = jnp.pad(X, ((0, NPAD - N), (0, 0)))

    t = _table_call(xpad, W1, _bcat(b1, H1), None, 128, H1)
    agg = _edge_combined_call(t, d32, s32, z128, H1, 80, 25, 5,
                              feature_split=False, with_ones=True)
    deg = agg[:, :, H1:H1 + DEG_W]

    t2 = _table_call(agg, W2, _bcat(b2, H2), deg, H1, H2, out_mode="pc64")
    agg = _edge_combined_call(t2, d16, s16, z128, H1, 80, 25, 10,
                              feature_split=True, with_ones=False)

    p3, q3 = _table_call(agg, W3, _bcat(b3, H3), deg, H2, H3,
                         out_mode="pc128", x_mode="cat")
    agg = _edge_split_call(p3, q3, d16, s16, z128)

    out = _head_call(
        agg, deg, Wf1, bf1.reshape(1, -1),
        Wf2.reshape(1, -1), bf2.reshape(1, 1),
    )
    return out[:N, 0]
